# Initial kernel scaffold; baseline (speedup 1.0000x reference)
import jax
import jax.numpy as jnp
from jax.experimental import pallas as pl


def kernel(x, edge_index, W_in, b_in, conv_w, env_Wlocal, env_mlp_w1, env_mlp_b1, env_mlp_w2, env_mlp_b2, env_alpha, env_fc_w, env_fc_b, W_out, b_out):
    raise NotImplementedError("write your pallas kernel here")


# speedup vs baseline: 5.8416x; 5.8416x over previous
import functools

import jax
import jax.numpy as jnp
from jax import lax
from jax.experimental import pallas as pl
from jax.experimental.pallas import tpu as pltpu
from jax.experimental.pallas import tpu_sc as plsc

N = 10000
E = 320000
F = 128
K = 3
L = 2

NC = 2
NS = 16
NW = NC * NS
CHUNK = 128
EPT = 10240
E_PAD = NW * EPT
N_ACC = 10240
RPT = N_ACC // NS
DW = 16

BN = 1000
GRID = N // BN

_sc_mesh = plsc.VectorSubcoreMesh(core_axis_name="c", subcore_axis_name="s")


def _deg_body(col_hbm, out_hbm, idx_v, ones_v, tmp_v, acc_sh, sem):
    c = lax.axis_index("c")
    s = lax.axis_index("s")
    wid = c * NS + s
    for r in range(CHUNK):
        for j in range(F // 16):
            ones_v[r, pl.ds(j * 16, 16)] = jnp.ones((16,), jnp.float32)
    for r in range(16):
        for j in range(F // 16):
            tmp_v[r, pl.ds(j * 16, 16)] = jnp.zeros((16,), jnp.float32)
    row0 = s * RPT

    def zbody(i, _):
        pltpu.sync_copy(tmp_v, acc_sh.at[pl.ds(row0 + i * 16, 16)])
        return 0

    lax.fori_loop(0, RPT // 16, zbody, 0, unroll=False)
    plsc.subcore_barrier()

    ebase = wid * EPT

    def cbody(i, _):
        pltpu.sync_copy(col_hbm.at[pl.ds(ebase + i * CHUNK, CHUNK)], idx_v.at[0])
        pltpu.sync_copy(ones_v, acc_sh.at[idx_v.at[0]], add=True)
        return 0

    lax.fori_loop(0, EPT // CHUNK, cbody, 0, unroll=False)
    plsc.subcore_barrier()

    obase = c * N_ACC + row0

    def fbody(i, _):
        pltpu.sync_copy(acc_sh.at[pl.ds(row0 + i * 16, 16)], tmp_v)
        pltpu.sync_copy(tmp_v, out_hbm.at[pl.ds(obase + i * 16, 16)])
        return 0

    lax.fori_loop(0, RPT // 16, fbody, 0, unroll=False)


_deg_call = pl.kernel(
    _deg_body,
    out_type=jax.ShapeDtypeStruct((NC * N_ACC, F), jnp.float32),
    mesh=_sc_mesh,
    scratch_types=[
        pltpu.VMEM((1, CHUNK), jnp.int32),
        pltpu.VMEM((CHUNK, F), jnp.float32),
        pltpu.VMEM((16, F), jnp.float32),
        pltpu.VMEM_SHARED((N_ACC, F), jnp.float32),
        pltpu.SemaphoreType.DMA,
    ],
)


def _spmv_body(row_hbm, col_hbm, g_hbm, out_hbm,
               idx_row_v, idx_col_v, rows_v, tmp_v, acc_sh, sem):
    c = lax.axis_index("c")
    s = lax.axis_index("s")
    wid = c * NS + s
    for r in range(16):
        for j in range(F // 16):
            tmp_v[r, pl.ds(j * 16, 16)] = jnp.zeros((16,), jnp.float32)
    row0 = s * RPT

    def zbody(i, _):
        pltpu.sync_copy(tmp_v, acc_sh.at[pl.ds(row0 + i * 16, 16)])
        return 0

    lax.fori_loop(0, RPT // 16, zbody, 0, unroll=False)
    plsc.subcore_barrier()

    ebase = wid * EPT

    def cbody(i, _):
        base = ebase + i * CHUNK
        pltpu.sync_copy(row_hbm.at[pl.ds(base, CHUNK)], idx_row_v.at[0])
        pltpu.sync_copy(col_hbm.at[pl.ds(base, CHUNK)], idx_col_v.at[0])
        pltpu.async_copy(g_hbm.at[idx_row_v.at[0]], rows_v, sem).wait()
        pltpu.sync_copy(rows_v, acc_sh.at[idx_col_v.at[0]], add=True)
        return 0

    lax.fori_loop(0, EPT // CHUNK, cbody, 0, unroll=False)
    plsc.subcore_barrier()

    obase = c * N_ACC + row0

    def fbody(i, _):
        pltpu.sync_copy(acc_sh.at[pl.ds(row0 + i * 16, 16)], tmp_v)
        pltpu.sync_copy(tmp_v, out_hbm.at[pl.ds(obase + i * 16, 16)])
        return 0

    lax.fori_loop(0, RPT // 16, fbody, 0, unroll=False)


_spmv_call = pl.kernel(
    _spmv_body,
    out_type=jax.ShapeDtypeStruct((NC * N_ACC, F), jnp.float32),
    mesh=_sc_mesh,
    scratch_types=[
        pltpu.VMEM((1, CHUNK), jnp.int32),
        pltpu.VMEM((1, CHUNK), jnp.int32),
        pltpu.VMEM((CHUNK, F), jnp.float32),
        pltpu.VMEM((16, F), jnp.float32),
        pltpu.VMEM_SHARED((N_ACC, F), jnp.float32),
        pltpu.SemaphoreType.DMA,
    ],
)


def _scale(d0_ref, d1_ref):
    d = d0_ref[:, 0:1] + d1_ref[:, 0:1]
    return jnp.where(d > 0.0, lax.rsqrt(jnp.maximum(d, 1e-30)), 0.0)


def _in_body(x_ref, w_ref, b_ref, d0_ref, d1_ref, h_ref, g_ref, cs_ref):
    h = jnp.maximum(jnp.dot(x_ref[...], w_ref[...],
                            preferred_element_type=jnp.float32) + b_ref[...], 0.0)
    h_ref[...] = h
    g_ref[...] = h * _scale(d0_ref, d1_ref)

    @pl.when(pl.program_id(0) == 0)
    def _():
        cs_ref[...] = jnp.zeros_like(cs_ref)

    cs_ref[...] += jnp.sum(h, axis=0, keepdims=True)


_in_call = pl.pallas_call(
    _in_body,
    grid=(GRID,),
    in_specs=[
        pl.BlockSpec((BN, F), lambda i: (i, 0)),
        pl.BlockSpec((F, F), lambda i: (0, 0)),
        pl.BlockSpec((1, F), lambda i: (0, 0)),
        pl.BlockSpec((BN, F), lambda i: (i, 0)),
        pl.BlockSpec((BN, F), lambda i: (i, 0)),
    ],
    out_specs=[
        pl.BlockSpec((BN, F), lambda i: (i, 0)),
        pl.BlockSpec((BN, F), lambda i: (i, 0)),
        pl.BlockSpec((1, F), lambda i: (0, 0)),
    ],
    out_shape=[
        jax.ShapeDtypeStruct((N, F), jnp.float32),
        jax.ShapeDtypeStruct((N, F), jnp.float32),
        jax.ShapeDtypeStruct((1, F), jnp.float32),
    ],
)


def _layer_body(is_last, h_ref, a0_ref, a1_ref, d0_ref, d1_ref, cs_ref,
                wl_ref, fcw_ref, fcb_ref, w1_ref, b1_ref, w2_ref, b2_ref,
                alpha_ref, conv_ref, wo_ref, bo_ref, *outs):
    s = _scale(d0_ref, d1_ref)
    hi = (a0_ref[...] + a1_ref[...]) * s
    h = h_ref[...]

    gp = cs_ref[...] * (1.0 / N)
    ge = jnp.dot(jnp.maximum(jnp.dot(gp, w1_ref[...],
                                     preferred_element_type=jnp.float32)
                             + b1_ref[...], 0.0),
                 w2_ref[...], preferred_element_type=jnp.float32) + b2_ref[...]
    wgt = jax.nn.sigmoid(alpha_ref[0, 0])
    m = jnp.dot(wl_ref[...], fcw_ref[...], preferred_element_type=jnp.float32)
    logits = (wgt * jnp.dot(hi, m, preferred_element_type=jnp.float32)
              + ((1.0 - wgt) * jnp.dot(ge, fcw_ref[...],
                                       preferred_element_type=jnp.float32)
                 + fcb_ref[...]))
    e = jax.nn.softmax(logits, axis=-1)

    hcat = jnp.concatenate([hi, h], axis=1)
    outs_all = jnp.dot(hcat, conv_ref[...], preferred_element_type=jnp.float32)
    out = h
    for k in range(K):
        out = out + e[:, k:k + 1] * outs_all[:, k * F:(k + 1) * F]
    hn = jnp.maximum(out, 0.0)

    if is_last:
        outs[0][...] = jnp.dot(hn, wo_ref[...],
                               preferred_element_type=jnp.float32) + bo_ref[...]
    else:
        outs[0][...] = hn
        outs[1][...] = hn * s

        @pl.when(pl.program_id(0) == 0)
        def _():
            outs[2][...] = jnp.zeros_like(outs[2])

        outs[2][...] += jnp.sum(hn, axis=0, keepdims=True)


def _make_layer_call(is_last):
    full = lambda i: (0, 0)
    in_specs = [
        pl.BlockSpec((BN, F), lambda i: (i, 0)),
        pl.BlockSpec((BN, F), lambda i: (i, 0)),
        pl.BlockSpec((BN, F), lambda i: (i, 0)),
        pl.BlockSpec((BN, F), lambda i: (i, 0)),
        pl.BlockSpec((BN, F), lambda i: (i, 0)),
        pl.BlockSpec((1, F), full),
        pl.BlockSpec((F, F), full),
        pl.BlockSpec((F, K), full),
        pl.BlockSpec((1, K), full),
        pl.BlockSpec((F, F), full),
        pl.BlockSpec((1, F), full),
        pl.BlockSpec((F, F), full),
        pl.BlockSpec((1, F), full),
        pl.BlockSpec((1, 1), full),
        pl.BlockSpec((2 * F, K * F), full),
        pl.BlockSpec((F, F), full),
        pl.BlockSpec((1, F), full),
    ]
    if is_last:
        out_specs = [pl.BlockSpec((BN, F), lambda i: (i, 0))]
        out_shape = [jax.ShapeDtypeStruct((N, F), jnp.float32)]
    else:
        out_specs = [
            pl.BlockSpec((BN, F), lambda i: (i, 0)),
            pl.BlockSpec((BN, F), lambda i: (i, 0)),
            pl.BlockSpec((1, F), full),
        ]
        out_shape = [
            jax.ShapeDtypeStruct((N, F), jnp.float32),
            jax.ShapeDtypeStruct((N, F), jnp.float32),
            jax.ShapeDtypeStruct((1, F), jnp.float32),
        ]
    return pl.pallas_call(
        functools.partial(_layer_body, is_last),
        grid=(GRID,),
        in_specs=in_specs,
        out_specs=out_specs,
        out_shape=out_shape,
    )


_layer_call = _make_layer_call(False)
_last_call = _make_layer_call(True)


def kernel(x, edge_index, W_in, b_in, conv_w, env_Wlocal, env_mlp_w1, env_mlp_b1,
           env_mlp_w2, env_mlp_b2, env_alpha, env_fc_w, env_fc_b, W_out, b_out):
    row = edge_index[0].astype(jnp.int32)
    col = edge_index[1].astype(jnp.int32)
    pad = E_PAD - E
    row_p = jnp.concatenate([row, jnp.zeros((pad,), jnp.int32)])
    col_p = jnp.concatenate([col, jnp.full((pad,), N, jnp.int32)])

    deg = _deg_call(col_p)
    d0 = deg[:N]
    d1 = deg[N_ACC:N_ACC + N]

    b_in2 = b_in.reshape(1, F)
    b_out2 = b_out.reshape(1, F)

    h, g, cs = _in_call(x, W_in, b_in2, d0, d1)

    for l in range(L):
        acc = _spmv_call(row_p, col_p, g)
        a0 = acc[:N]
        a1 = acc[N_ACC:N_ACC + N]
        conv_r = conv_w[l].transpose(1, 0, 2).reshape(2 * F, K * F)
        args = (h, a0, a1, d0, d1, cs,
                env_Wlocal[l], env_fc_w[l], env_fc_b[l].reshape(1, K),
                env_mlp_w1[l], env_mlp_b1[l].reshape(1, F),
                env_mlp_w2[l], env_mlp_b2[l].reshape(1, F),
                env_alpha[l].reshape(1, 1), conv_r, W_out, b_out2)
        if l == L - 1:
            out = _last_call(*args)[0]
        else:
            h, g, cs = _layer_call(*args)
    return out


# speedup vs baseline: 6.1977x; 1.0610x over previous
import functools

import jax
import jax.numpy as jnp
from jax import lax
from jax.experimental import pallas as pl
from jax.experimental.pallas import tpu as pltpu
from jax.experimental.pallas import tpu_sc as plsc

N = 10000
E = 320000
F = 128
K = 3
L = 2

NC = 2
NS = 16
NW = NC * NS
CHUNK = 128
EPT = 10240
E_PAD = NW * EPT
N_ACC = 10240
RPT = N_ACC // NS
DW = 16

BN = 1000
GRID = N // BN

_sc_mesh = plsc.VectorSubcoreMesh(core_axis_name="c", subcore_axis_name="s")


def _deg_body(col_hbm, out_hbm, idx_v, ones_v, tmp_v, acc_sh, sem):
    c = lax.axis_index("c")
    s = lax.axis_index("s")
    wid = c * NS + s
    for r in range(CHUNK):
        for j in range(F // 16):
            ones_v[r, pl.ds(j * 16, 16)] = jnp.ones((16,), jnp.float32)
    for r in range(16):
        for j in range(F // 16):
            tmp_v[r, pl.ds(j * 16, 16)] = jnp.zeros((16,), jnp.float32)
    row0 = s * RPT

    def zbody(i, _):
        pltpu.sync_copy(tmp_v, acc_sh.at[pl.ds(row0 + i * 16, 16)])
        return 0

    lax.fori_loop(0, RPT // 16, zbody, 0, unroll=False)
    plsc.subcore_barrier()

    ebase = wid * EPT

    def cbody(i, _):
        pltpu.sync_copy(col_hbm.at[pl.ds(ebase + i * CHUNK, CHUNK)], idx_v.at[0])
        pltpu.sync_copy(ones_v, acc_sh.at[idx_v.at[0]], add=True)
        return 0

    lax.fori_loop(0, EPT // CHUNK, cbody, 0, unroll=False)
    plsc.subcore_barrier()

    obase = c * N_ACC + row0

    def fbody(i, _):
        pltpu.sync_copy(acc_sh.at[pl.ds(row0 + i * 16, 16)], tmp_v)
        pltpu.sync_copy(tmp_v, out_hbm.at[pl.ds(obase + i * 16, 16)])
        return 0

    lax.fori_loop(0, RPT // 16, fbody, 0, unroll=False)


_deg_call = pl.kernel(
    _deg_body,
    out_type=jax.ShapeDtypeStruct((NC * N_ACC, F), jnp.float32),
    mesh=_sc_mesh,
    scratch_types=[
        pltpu.VMEM((1, CHUNK), jnp.int32),
        pltpu.VMEM((CHUNK, F), jnp.float32),
        pltpu.VMEM((16, F), jnp.float32),
        pltpu.VMEM_SHARED((N_ACC, F), jnp.float32),
        pltpu.SemaphoreType.DMA,
    ],
)


NCH = EPT // CHUNK


def _spmv_body(row_hbm, col_hbm, g_hbm, out_hbm,
               idx_row_v, idx_col_v, rows0_v, rows1_v, tmp_v, acc_sh,
               sem0, sem1, semi):
    c = lax.axis_index("c")
    s = lax.axis_index("s")
    wid = c * NS + s
    for r in range(16):
        for j in range(F // 16):
            tmp_v[r, pl.ds(j * 16, 16)] = jnp.zeros((16,), jnp.float32)
    row0 = s * RPT

    gc0 = wid * NCH

    def zbody(i, _):
        pltpu.sync_copy(tmp_v, acc_sh.at[pl.ds(row0 + i * 16, 16)])
        return 0

    lax.fori_loop(0, RPT // 16, zbody, 0, unroll=False)

    pltpu.sync_copy(row_hbm.at[pl.ds(gc0, 1)], idx_row_v.at[pl.ds(0, 1)])
    pltpu.sync_copy(col_hbm.at[pl.ds(gc0, 1)], idx_col_v.at[pl.ds(0, 1)])
    plsc.subcore_barrier()
    pltpu.async_copy(g_hbm.at[idx_row_v.at[0]], rows0_v, sem0)
    pltpu.async_copy(row_hbm.at[pl.ds(gc0 + 1, 1)],
                     idx_row_v.at[pl.ds(1, 1)], semi)
    pltpu.async_copy(col_hbm.at[pl.ds(gc0 + 1, 1)],
                     idx_col_v.at[pl.ds(1, 1)], semi)

    def _wait_idx(i):
        pltpu.make_async_copy(row_hbm.at[pl.ds(gc0 + i, 1)],
                              idx_row_v.at[pl.ds(1, 1)], semi).wait()
        pltpu.make_async_copy(col_hbm.at[pl.ds(gc0 + i, 1)],
                              idx_col_v.at[pl.ds(1, 1)], semi).wait()

    def cbody(j, _):
        i0 = 2 * j
        _wait_idx(i0 + 1)
        pltpu.async_copy(g_hbm.at[idx_row_v.at[1]], rows1_v, sem1)
        pltpu.make_async_copy(g_hbm.at[idx_row_v.at[0]], rows0_v, sem0).wait()
        pltpu.sync_copy(rows0_v, acc_sh.at[idx_col_v.at[0]], add=True)

        @pl.when(i0 + 2 < NCH)
        def _():
            pltpu.sync_copy(row_hbm.at[pl.ds(gc0 + i0 + 2, 1)],
                            idx_row_v.at[pl.ds(0, 1)])
            pltpu.sync_copy(col_hbm.at[pl.ds(gc0 + i0 + 2, 1)],
                            idx_col_v.at[pl.ds(0, 1)])
            pltpu.async_copy(g_hbm.at[idx_row_v.at[0]], rows0_v, sem0)

        pltpu.make_async_copy(g_hbm.at[idx_row_v.at[1]], rows1_v, sem1).wait()
        pltpu.sync_copy(rows1_v, acc_sh.at[idx_col_v.at[1]], add=True)

        @pl.when(i0 + 3 < NCH)
        def _():
            pltpu.async_copy(row_hbm.at[pl.ds(gc0 + i0 + 3, 1)],
                             idx_row_v.at[pl.ds(1, 1)], semi)
            pltpu.async_copy(col_hbm.at[pl.ds(gc0 + i0 + 3, 1)],
                             idx_col_v.at[pl.ds(1, 1)], semi)
        return 0

    lax.fori_loop(0, NCH // 2, cbody, 0, unroll=False)
    plsc.subcore_barrier()

    obase = c * N_ACC + row0

    def fbody(i, _):
        pltpu.sync_copy(acc_sh.at[pl.ds(row0 + i * 16, 16)], tmp_v)
        pltpu.sync_copy(tmp_v, out_hbm.at[pl.ds(obase + i * 16, 16)])
        return 0

    lax.fori_loop(0, RPT // 16, fbody, 0, unroll=False)


_spmv_call = pl.kernel(
    _spmv_body,
    out_type=jax.ShapeDtypeStruct((NC * N_ACC, F), jnp.float32),
    mesh=_sc_mesh,
    scratch_types=[
        pltpu.VMEM((2, CHUNK), jnp.int32),
        pltpu.VMEM((2, CHUNK), jnp.int32),
        pltpu.VMEM((CHUNK, F), jnp.float32),
        pltpu.VMEM((CHUNK, F), jnp.float32),
        pltpu.VMEM((16, F), jnp.float32),
        pltpu.VMEM_SHARED((N_ACC, F), jnp.float32),
        pltpu.SemaphoreType.DMA,
        pltpu.SemaphoreType.DMA,
        pltpu.SemaphoreType.DMA,
    ],
)


def _scale(d0_ref, d1_ref):
    d = d0_ref[:, 0:1] + d1_ref[:, 0:1]
    return jnp.where(d > 0.0, lax.rsqrt(jnp.maximum(d, 1e-30)), 0.0)


def _in_body(x_ref, w_ref, b_ref, d0_ref, d1_ref, h_ref, g_ref, cs_ref):
    h = jnp.maximum(jnp.dot(x_ref[...], w_ref[...],
                            preferred_element_type=jnp.float32) + b_ref[...], 0.0)
    h_ref[...] = h
    g_ref[...] = h * _scale(d0_ref, d1_ref)

    @pl.when(pl.program_id(0) == 0)
    def _():
        cs_ref[...] = jnp.zeros_like(cs_ref)

    cs_ref[...] += jnp.sum(h, axis=0, keepdims=True)


_in_call = pl.pallas_call(
    _in_body,
    grid=(GRID,),
    in_specs=[
        pl.BlockSpec((BN, F), lambda i: (i, 0)),
        pl.BlockSpec((F, F), lambda i: (0, 0)),
        pl.BlockSpec((1, F), lambda i: (0, 0)),
        pl.BlockSpec((BN, F), lambda i: (i, 0)),
        pl.BlockSpec((BN, F), lambda i: (i, 0)),
    ],
    out_specs=[
        pl.BlockSpec((BN, F), lambda i: (i, 0)),
        pl.BlockSpec((BN, F), lambda i: (i, 0)),
        pl.BlockSpec((1, F), lambda i: (0, 0)),
    ],
    out_shape=[
        jax.ShapeDtypeStruct((N, F), jnp.float32),
        jax.ShapeDtypeStruct((N, F), jnp.float32),
        jax.ShapeDtypeStruct((1, F), jnp.float32),
    ],
)


def _layer_body(is_last, h_ref, a0_ref, a1_ref, d0_ref, d1_ref, cs_ref,
                wl_ref, fcw_ref, fcb_ref, w1_ref, b1_ref, w2_ref, b2_ref,
                alpha_ref, conv_ref, wo_ref, bo_ref, *outs):
    s = _scale(d0_ref, d1_ref)
    hi = (a0_ref[...] + a1_ref[...]) * s
    h = h_ref[...]

    gp = cs_ref[...] * (1.0 / N)
    ge = jnp.dot(jnp.maximum(jnp.dot(gp, w1_ref[...],
                                     preferred_element_type=jnp.float32)
                             + b1_ref[...], 0.0),
                 w2_ref[...], preferred_element_type=jnp.float32) + b2_ref[...]
    wgt = jax.nn.sigmoid(alpha_ref[0, 0])
    m = jnp.dot(wl_ref[...], fcw_ref[...], preferred_element_type=jnp.float32)
    logits = (wgt * jnp.dot(hi, m, preferred_element_type=jnp.float32)
              + ((1.0 - wgt) * jnp.dot(ge, fcw_ref[...],
                                       preferred_element_type=jnp.float32)
                 + fcb_ref[...]))
    e = jax.nn.softmax(logits, axis=-1)

    hcat = jnp.concatenate([hi, h], axis=1)
    outs_all = jnp.dot(hcat, conv_ref[...], preferred_element_type=jnp.float32)
    out = h
    for k in range(K):
        out = out + e[:, k:k + 1] * outs_all[:, k * F:(k + 1) * F]
    hn = jnp.maximum(out, 0.0)

    if is_last:
        outs[0][...] = jnp.dot(hn, wo_ref[...],
                               preferred_element_type=jnp.float32) + bo_ref[...]
    else:
        outs[0][...] = hn
        outs[1][...] = hn * s

        @pl.when(pl.program_id(0) == 0)
        def _():
            outs[2][...] = jnp.zeros_like(outs[2])

        outs[2][...] += jnp.sum(hn, axis=0, keepdims=True)


def _make_layer_call(is_last):
    full = lambda i: (0, 0)
    in_specs = [
        pl.BlockSpec((BN, F), lambda i: (i, 0)),
        pl.BlockSpec((BN, F), lambda i: (i, 0)),
        pl.BlockSpec((BN, F), lambda i: (i, 0)),
        pl.BlockSpec((BN, F), lambda i: (i, 0)),
        pl.BlockSpec((BN, F), lambda i: (i, 0)),
        pl.BlockSpec((1, F), full),
        pl.BlockSpec((F, F), full),
        pl.BlockSpec((F, K), full),
        pl.BlockSpec((1, K), full),
        pl.BlockSpec((F, F), full),
        pl.BlockSpec((1, F), full),
        pl.BlockSpec((F, F), full),
        pl.BlockSpec((1, F), full),
        pl.BlockSpec((1, 1), full),
        pl.BlockSpec((2 * F, K * F), full),
        pl.BlockSpec((F, F), full),
        pl.BlockSpec((1, F), full),
    ]
    if is_last:
        out_specs = [pl.BlockSpec((BN, F), lambda i: (i, 0))]
        out_shape = [jax.ShapeDtypeStruct((N, F), jnp.float32)]
    else:
        out_specs = [
            pl.BlockSpec((BN, F), lambda i: (i, 0)),
            pl.BlockSpec((BN, F), lambda i: (i, 0)),
            pl.BlockSpec((1, F), full),
        ]
        out_shape = [
            jax.ShapeDtypeStruct((N, F), jnp.float32),
            jax.ShapeDtypeStruct((N, F), jnp.float32),
            jax.ShapeDtypeStruct((1, F), jnp.float32),
        ]
    return pl.pallas_call(
        functools.partial(_layer_body, is_last),
        grid=(GRID,),
        in_specs=in_specs,
        out_specs=out_specs,
        out_shape=out_shape,
    )


_layer_call = _make_layer_call(False)
_last_call = _make_layer_call(True)


def kernel(x, edge_index, W_in, b_in, conv_w, env_Wlocal, env_mlp_w1, env_mlp_b1,
           env_mlp_w2, env_mlp_b2, env_alpha, env_fc_w, env_fc_b, W_out, b_out):
    row = edge_index[0].astype(jnp.int32)
    col = edge_index[1].astype(jnp.int32)
    pad = E_PAD - E
    row_p = jnp.concatenate([row, jnp.zeros((pad,), jnp.int32)])
    col_p = jnp.concatenate([col, jnp.full((pad,), N, jnp.int32)])

    row_2d = row_p.reshape(E_PAD // CHUNK, CHUNK)
    col_2d = col_p.reshape(E_PAD // CHUNK, CHUNK)

    deg = _deg_call(col_p)
    d0 = deg[:N]
    d1 = deg[N_ACC:N_ACC + N]

    b_in2 = b_in.reshape(1, F)
    b_out2 = b_out.reshape(1, F)

    h, g, cs = _in_call(x, W_in, b_in2, d0, d1)

    for l in range(L):
        acc = _spmv_call(row_2d, col_2d, g)
        a0 = acc[:N]
        a1 = acc[N_ACC:N_ACC + N]
        conv_r = conv_w[l].transpose(1, 0, 2).reshape(2 * F, K * F)
        args = (h, a0, a1, d0, d1, cs,
                env_Wlocal[l], env_fc_w[l], env_fc_b[l].reshape(1, K),
                env_mlp_w1[l], env_mlp_b1[l].reshape(1, F),
                env_mlp_w2[l], env_mlp_b2[l].reshape(1, F),
                env_alpha[l].reshape(1, 1), conv_r, W_out, b_out2)
        if l == L - 1:
            out = _last_call(*args)[0]
        else:
            h, g, cs = _layer_call(*args)
    return out


# speedup vs baseline: 15.3816x; 2.4818x over previous
import functools

import jax
import jax.numpy as jnp
from jax import lax
from jax.experimental import pallas as pl
from jax.experimental.pallas import tpu as pltpu
from jax.experimental.pallas import tpu_sc as plsc

N = 10000
E = 320000
F = 128
K = 3
L = 2

NC = 2
NS = 16
NW = NC * NS
CHUNK = 128
EPT = 10240
E_PAD = NW * EPT
N_ACC = 10240
RPT = N_ACC // NS
DW = 16

BN = 1000
GRID = N // BN

_sc_mesh = plsc.VectorSubcoreMesh(core_axis_name="c", subcore_axis_name="s")


def _deg_body(col_hbm, out_hbm, idx_v, ones_v, tmp_v, acc_sh, sem):
    c = lax.axis_index("c")
    s = lax.axis_index("s")
    wid = c * NS + s
    for r in range(CHUNK):
        for j in range(F // 16):
            ones_v[r, pl.ds(j * 16, 16)] = jnp.ones((16,), jnp.float32)
    for r in range(16):
        for j in range(F // 16):
            tmp_v[r, pl.ds(j * 16, 16)] = jnp.zeros((16,), jnp.float32)
    row0 = s * RPT

    def zbody(i, _):
        pltpu.sync_copy(tmp_v, acc_sh.at[pl.ds(row0 + i * 16, 16)])
        return 0

    lax.fori_loop(0, RPT // 16, zbody, 0, unroll=False)
    plsc.subcore_barrier()

    ebase = wid * EPT

    def cbody(i, _):
        pltpu.sync_copy(col_hbm.at[pl.ds(ebase + i * CHUNK, CHUNK)], idx_v.at[0])
        pltpu.sync_copy(ones_v, acc_sh.at[idx_v.at[0]], add=True)
        return 0

    lax.fori_loop(0, EPT // CHUNK, cbody, 0, unroll=False)
    plsc.subcore_barrier()

    obase = c * N_ACC + row0

    def fbody(i, _):
        pltpu.sync_copy(acc_sh.at[pl.ds(row0 + i * 16, 16)], tmp_v)
        pltpu.sync_copy(tmp_v, out_hbm.at[pl.ds(obase + i * 16, 16)])
        return 0

    lax.fori_loop(0, RPT // 16, fbody, 0, unroll=False)


_deg_call = pl.kernel(
    _deg_body,
    out_type=jax.ShapeDtypeStruct((NC * N_ACC, F), jnp.float32),
    mesh=_sc_mesh,
    scratch_types=[
        pltpu.VMEM((1, CHUNK), jnp.int32),
        pltpu.VMEM((CHUNK, F), jnp.float32),
        pltpu.VMEM((16, F), jnp.float32),
        pltpu.VMEM_SHARED((N_ACC, F), jnp.float32),
        pltpu.SemaphoreType.DMA,
    ],
)


NCH = EPT // CHUNK


def _spmv_body(row_hbm, col_hbm, g_hbm, out_hbm,
               idx_row_v, idx_col_v, rows0_v, rows1_v, tmp_v, acc_sh,
               sem0, sem1, semi):
    c = lax.axis_index("c")
    s = lax.axis_index("s")
    wid = c * NS + s
    for r in range(16):
        for j in range(F // 16):
            tmp_v[r, pl.ds(j * 16, 16)] = jnp.zeros((16,), jnp.float32)
    row0 = s * RPT

    gc0 = wid * NCH

    def zbody(i, _):
        pltpu.sync_copy(tmp_v, acc_sh.at[pl.ds(row0 + i * 16, 16)])
        return 0

    lax.fori_loop(0, RPT // 16, zbody, 0, unroll=False)

    pltpu.sync_copy(row_hbm.at[pl.ds(gc0, 1)], idx_row_v.at[pl.ds(0, 1)])
    pltpu.sync_copy(col_hbm.at[pl.ds(gc0, 1)], idx_col_v.at[pl.ds(0, 1)])
    plsc.subcore_barrier()
    pltpu.async_copy(g_hbm.at[idx_row_v.at[0]], rows0_v, sem0)
    pltpu.async_copy(row_hbm.at[pl.ds(gc0 + 1, 1)],
                     idx_row_v.at[pl.ds(1, 1)], semi)
    pltpu.async_copy(col_hbm.at[pl.ds(gc0 + 1, 1)],
                     idx_col_v.at[pl.ds(1, 1)], semi)

    def _wait_idx(i):
        pltpu.make_async_copy(row_hbm.at[pl.ds(gc0 + i, 1)],
                              idx_row_v.at[pl.ds(1, 1)], semi).wait()
        pltpu.make_async_copy(col_hbm.at[pl.ds(gc0 + i, 1)],
                              idx_col_v.at[pl.ds(1, 1)], semi).wait()

    def cbody(j, _):
        i0 = 2 * j
        _wait_idx(i0 + 1)
        pltpu.async_copy(g_hbm.at[idx_row_v.at[1]], rows1_v, sem1)
        pltpu.make_async_copy(g_hbm.at[idx_row_v.at[0]], rows0_v, sem0).wait()
        pltpu.sync_copy(rows0_v, acc_sh.at[idx_col_v.at[0]], add=True)

        @pl.when(i0 + 2 < NCH)
        def _():
            pltpu.sync_copy(row_hbm.at[pl.ds(gc0 + i0 + 2, 1)],
                            idx_row_v.at[pl.ds(0, 1)])
            pltpu.sync_copy(col_hbm.at[pl.ds(gc0 + i0 + 2, 1)],
                            idx_col_v.at[pl.ds(0, 1)])
            pltpu.async_copy(g_hbm.at[idx_row_v.at[0]], rows0_v, sem0)

        pltpu.make_async_copy(g_hbm.at[idx_row_v.at[1]], rows1_v, sem1).wait()
        pltpu.sync_copy(rows1_v, acc_sh.at[idx_col_v.at[1]], add=True)

        @pl.when(i0 + 3 < NCH)
        def _():
            pltpu.async_copy(row_hbm.at[pl.ds(gc0 + i0 + 3, 1)],
                             idx_row_v.at[pl.ds(1, 1)], semi)
            pltpu.async_copy(col_hbm.at[pl.ds(gc0 + i0 + 3, 1)],
                             idx_col_v.at[pl.ds(1, 1)], semi)
        return 0

    lax.fori_loop(0, NCH // 2, cbody, 0, unroll=False)
    plsc.subcore_barrier()

    obase = c * N_ACC + row0

    def fbody(i, _):
        pltpu.sync_copy(acc_sh.at[pl.ds(row0 + i * 16, 16)], tmp_v)
        pltpu.sync_copy(tmp_v, out_hbm.at[pl.ds(obase + i * 16, 16)])
        return 0

    lax.fori_loop(0, RPT // 16, fbody, 0, unroll=False)


_spmv_call = pl.kernel(
    _spmv_body,
    out_type=jax.ShapeDtypeStruct((NC * N_ACC, F), jnp.float32),
    mesh=_sc_mesh,
    scratch_types=[
        pltpu.VMEM((2, CHUNK), jnp.int32),
        pltpu.VMEM((2, CHUNK), jnp.int32),
        pltpu.VMEM((CHUNK, F), jnp.float32),
        pltpu.VMEM((CHUNK, F), jnp.float32),
        pltpu.VMEM((16, F), jnp.float32),
        pltpu.VMEM_SHARED((N_ACC, F), jnp.float32),
        pltpu.SemaphoreType.DMA,
        pltpu.SemaphoreType.DMA,
        pltpu.SemaphoreType.DMA,
    ],
)


def _scale(d0_ref, d1_ref):
    d = d0_ref[:, 0:1] + d1_ref[:, 0:1]
    return jnp.where(d > 0.0, lax.rsqrt(jnp.maximum(d, 1e-30)), 0.0)


def _in_body(x_ref, w_ref, b_ref, d0_ref, d1_ref, h_ref, g_ref, cs_ref):
    h = jnp.maximum(jnp.dot(x_ref[...], w_ref[...],
                            preferred_element_type=jnp.float32) + b_ref[...], 0.0)
    h_ref[...] = h
    g_ref[...] = h * _scale(d0_ref, d1_ref)

    @pl.when(pl.program_id(0) == 0)
    def _():
        cs_ref[...] = jnp.zeros_like(cs_ref)

    cs_ref[...] += jnp.sum(h, axis=0, keepdims=True)


_in_call = pl.pallas_call(
    _in_body,
    grid=(GRID,),
    in_specs=[
        pl.BlockSpec((BN, F), lambda i: (i, 0)),
        pl.BlockSpec((F, F), lambda i: (0, 0)),
        pl.BlockSpec((1, F), lambda i: (0, 0)),
        pl.BlockSpec((BN, F), lambda i: (i, 0)),
        pl.BlockSpec((BN, F), lambda i: (i, 0)),
    ],
    out_specs=[
        pl.BlockSpec((BN, F), lambda i: (i, 0)),
        pl.BlockSpec((BN, F), lambda i: (i, 0)),
        pl.BlockSpec((1, F), lambda i: (0, 0)),
    ],
    out_shape=[
        jax.ShapeDtypeStruct((N, F), jnp.float32),
        jax.ShapeDtypeStruct((N, F), jnp.float32),
        jax.ShapeDtypeStruct((1, F), jnp.float32),
    ],
)


def _layer_body(is_last, h_ref, a0_ref, a1_ref, d0_ref, d1_ref, cs_ref,
                wl_ref, fcw_ref, fcb_ref, w1_ref, b1_ref, w2_ref, b2_ref,
                alpha_ref, conv_ref, wo_ref, bo_ref, *outs):
    s = _scale(d0_ref, d1_ref)
    hi = (a0_ref[...] + a1_ref[...]) * s
    h = h_ref[...]

    gp = cs_ref[...] * (1.0 / N)
    ge = jnp.dot(jnp.maximum(jnp.dot(gp, w1_ref[...],
                                     preferred_element_type=jnp.float32)
                             + b1_ref[...], 0.0),
                 w2_ref[...], preferred_element_type=jnp.float32) + b2_ref[...]
    wgt = jax.nn.sigmoid(alpha_ref[0, 0])
    m = jnp.dot(wl_ref[...], fcw_ref[...], preferred_element_type=jnp.float32)
    logits = (wgt * jnp.dot(hi, m, preferred_element_type=jnp.float32)
              + ((1.0 - wgt) * jnp.dot(ge, fcw_ref[...],
                                       preferred_element_type=jnp.float32)
                 + fcb_ref[...]))
    e = jax.nn.softmax(logits, axis=-1)

    hcat = jnp.concatenate([hi, h], axis=1)
    outs_all = jnp.dot(hcat, conv_ref[...], preferred_element_type=jnp.float32)
    out = h
    for k in range(K):
        out = out + e[:, k:k + 1] * outs_all[:, k * F:(k + 1) * F]
    hn = jnp.maximum(out, 0.0)

    if is_last:
        outs[0][...] = jnp.dot(hn, wo_ref[...],
                               preferred_element_type=jnp.float32) + bo_ref[...]
    else:
        outs[0][...] = hn
        outs[1][...] = hn * s

        @pl.when(pl.program_id(0) == 0)
        def _():
            outs[2][...] = jnp.zeros_like(outs[2])

        outs[2][...] += jnp.sum(hn, axis=0, keepdims=True)


def _make_layer_call(is_last):
    full = lambda i: (0, 0)
    in_specs = [
        pl.BlockSpec((BN, F), lambda i: (i, 0)),
        pl.BlockSpec((BN, F), lambda i: (i, 0)),
        pl.BlockSpec((BN, F), lambda i: (i, 0)),
        pl.BlockSpec((BN, F), lambda i: (i, 0)),
        pl.BlockSpec((BN, F), lambda i: (i, 0)),
        pl.BlockSpec((1, F), full),
        pl.BlockSpec((F, F), full),
        pl.BlockSpec((F, K), full),
        pl.BlockSpec((1, K), full),
        pl.BlockSpec((F, F), full),
        pl.BlockSpec((1, F), full),
        pl.BlockSpec((F, F), full),
        pl.BlockSpec((1, F), full),
        pl.BlockSpec((1, 1), full),
        pl.BlockSpec((2 * F, K * F), full),
        pl.BlockSpec((F, F), full),
        pl.BlockSpec((1, F), full),
    ]
    if is_last:
        out_specs = [pl.BlockSpec((BN, F), lambda i: (i, 0))]
        out_shape = [jax.ShapeDtypeStruct((N, F), jnp.float32)]
    else:
        out_specs = [
            pl.BlockSpec((BN, F), lambda i: (i, 0)),
            pl.BlockSpec((BN, F), lambda i: (i, 0)),
            pl.BlockSpec((1, F), full),
        ]
        out_shape = [
            jax.ShapeDtypeStruct((N, F), jnp.float32),
            jax.ShapeDtypeStruct((N, F), jnp.float32),
            jax.ShapeDtypeStruct((1, F), jnp.float32),
        ]
    return pl.pallas_call(
        functools.partial(_layer_body, is_last),
        grid=(GRID,),
        in_specs=in_specs,
        out_specs=out_specs,
        out_shape=out_shape,
    )


_layer_call = _make_layer_call(False)
_last_call = _make_layer_call(True)


def kernel(x, edge_index, W_in, b_in, conv_w, env_Wlocal, env_mlp_w1, env_mlp_b1,
           env_mlp_w2, env_mlp_b2, env_alpha, env_fc_w, env_fc_b, W_out, b_out):
    row = edge_index[0].astype(jnp.int32)
    col = edge_index[1].astype(jnp.int32)
    pad = E_PAD - E
    ar = jnp.arange(pad, dtype=jnp.int32)
    row_p = jnp.concatenate([row, (ar * 37) % N])
    col_p = jnp.concatenate([col, N + (ar % (N_ACC - N))])

    row_2d = row_p.reshape(E_PAD // CHUNK, CHUNK)
    col_2d = col_p.reshape(E_PAD // CHUNK, CHUNK)

    deg = _deg_call(col_p)
    d0 = deg[:N]
    d1 = deg[N_ACC:N_ACC + N]

    b_in2 = b_in.reshape(1, F)
    b_out2 = b_out.reshape(1, F)

    h, g, cs = _in_call(x, W_in, b_in2, d0, d1)

    for l in range(L):
        acc = _spmv_call(row_2d, col_2d, g)
        a0 = acc[:N]
        a1 = acc[N_ACC:N_ACC + N]
        conv_r = conv_w[l].transpose(1, 0, 2).reshape(2 * F, K * F)
        args = (h, a0, a1, d0, d1, cs,
                env_Wlocal[l], env_fc_w[l], env_fc_b[l].reshape(1, K),
                env_mlp_w1[l], env_mlp_b1[l].reshape(1, F),
                env_mlp_w2[l], env_mlp_b2[l].reshape(1, F),
                env_alpha[l].reshape(1, 1), conv_r, W_out, b_out2)
        if l == L - 1:
            out = _last_call(*args)[0]
        else:
            h, g, cs = _layer_call(*args)
    return out


# speedup vs baseline: 17.1876x; 1.1174x over previous
import functools

import jax
import jax.numpy as jnp
from jax import lax
from jax.experimental import pallas as pl
from jax.experimental.pallas import tpu as pltpu
from jax.experimental.pallas import tpu_sc as plsc

N = 10000
E = 320000
F = 128
K = 3
L = 2

NC = 2
NS = 16
NW = NC * NS
CHUNK = 128
EPT = 10240
E_PAD = NW * EPT
N_ACC = 10240
RPT = N_ACC // NS
DW = 16

BN = 1000
GRID = N // BN

_sc_mesh = plsc.VectorSubcoreMesh(core_axis_name="c", subcore_axis_name="s")


def _deg_body(col_hbm, out_hbm, idx_v, ones_v, tmp_v, acc_sh, sem):
    c = lax.axis_index("c")
    s = lax.axis_index("s")
    wid = c * NS + s
    for r in range(CHUNK):
        for j in range(F // 16):
            ones_v[r, pl.ds(j * 16, 16)] = jnp.ones((16,), jnp.float32)
    for r in range(16):
        for j in range(F // 16):
            tmp_v[r, pl.ds(j * 16, 16)] = jnp.zeros((16,), jnp.float32)
    row0 = s * RPT

    def zbody(i, _):
        pltpu.sync_copy(tmp_v, acc_sh.at[pl.ds(row0 + i * 16, 16)])
        return 0

    lax.fori_loop(0, RPT // 16, zbody, 0, unroll=False)
    plsc.subcore_barrier()

    ebase = wid * EPT

    def cbody(i, _):
        pltpu.sync_copy(col_hbm.at[pl.ds(ebase + i * CHUNK, CHUNK)], idx_v.at[0])
        pltpu.sync_copy(ones_v, acc_sh.at[idx_v.at[0]], add=True)
        return 0

    lax.fori_loop(0, EPT // CHUNK, cbody, 0, unroll=False)
    plsc.subcore_barrier()

    obase = c * N_ACC + row0

    def fbody(i, _):
        pltpu.sync_copy(acc_sh.at[pl.ds(row0 + i * 16, 16)], tmp_v)
        pltpu.sync_copy(tmp_v, out_hbm.at[pl.ds(obase + i * 16, 16)])
        return 0

    lax.fori_loop(0, RPT // 16, fbody, 0, unroll=False)


_deg_call = pl.kernel(
    _deg_body,
    out_type=jax.ShapeDtypeStruct((NC * N_ACC, F), jnp.float32),
    mesh=_sc_mesh,
    scratch_types=[
        pltpu.VMEM((1, CHUNK), jnp.int32),
        pltpu.VMEM((CHUNK, F), jnp.float32),
        pltpu.VMEM((16, F), jnp.float32),
        pltpu.VMEM_SHARED((N_ACC, F), jnp.float32),
        pltpu.SemaphoreType.DMA,
    ],
)


NCH = EPT // CHUNK


def _spmv_body(row_hbm, col_hbm, g_hbm, out_hbm,
               idx_row_v, idx_col_v, rows0_v, rows1_v, tmp_v, acc_sh,
               sem0, sem1, semi0, semi1, semsc0, semsc1):
    c = lax.axis_index("c")
    s = lax.axis_index("s")
    wid = c * NS + s
    for r in range(16):
        for j in range(F // 16):
            tmp_v[r, pl.ds(j * 16, 16)] = jnp.zeros((16,), jnp.float32)
    row0 = s * RPT

    gc0 = wid * NCH

    def zbody(i, _):
        pltpu.sync_copy(tmp_v, acc_sh.at[pl.ds(row0 + i * 16, 16)])
        return 0

    lax.fori_loop(0, RPT // 16, zbody, 0, unroll=False)

    def idx_load(i, rb, sem):
        pltpu.async_copy(row_hbm.at[pl.ds(gc0 + i, 1)],
                         idx_row_v.at[pl.ds(rb, 1)], sem)
        pltpu.async_copy(col_hbm.at[pl.ds(gc0 + i, 1)],
                         idx_col_v.at[pl.ds((gc0 + i) % 4, 1)], sem)

    def idx_wait(i, rb, sem):
        pltpu.make_async_copy(row_hbm.at[pl.ds(gc0 + i, 1)],
                              idx_row_v.at[pl.ds(rb, 1)], sem).wait()
        pltpu.make_async_copy(col_hbm.at[pl.ds(gc0 + i, 1)],
                              idx_col_v.at[pl.ds((gc0 + i) % 4, 1)], sem).wait()

    def scat_start(i, rows, sem):
        pltpu.async_copy(rows, acc_sh.at[idx_col_v.at[(gc0 + i) % 4]], sem,
                         add=True)

    def scat_wait(i, rows, sem):
        pltpu.make_async_copy(rows, acc_sh.at[idx_col_v.at[(gc0 + i) % 4]],
                              sem).wait()

    pltpu.sync_copy(row_hbm.at[pl.ds(gc0, 1)], idx_row_v.at[pl.ds(0, 1)])
    pltpu.sync_copy(col_hbm.at[pl.ds(gc0, 1)],
                    idx_col_v.at[pl.ds(gc0 % 4, 1)])
    plsc.subcore_barrier()
    pltpu.async_copy(g_hbm.at[idx_row_v.at[0]], rows0_v, sem0)
    idx_load(1, 1, semi1)

    def cbody(j, _):
        i0 = 2 * j
        idx_wait(i0 + 1, 1, semi1)

        @pl.when(j > 0)
        def _():
            scat_wait(i0 - 1, rows1_v, semsc1)
        pltpu.async_copy(g_hbm.at[idx_row_v.at[1]], rows1_v, sem1)

        pltpu.make_async_copy(g_hbm.at[idx_row_v.at[0]], rows0_v, sem0).wait()
        scat_start(i0, rows0_v, semsc0)

        @pl.when(i0 + 2 < NCH)
        def _():
            idx_load(i0 + 2, 0, semi0)

        pltpu.make_async_copy(g_hbm.at[idx_row_v.at[1]], rows1_v, sem1).wait()
        scat_wait(i0, rows0_v, semsc0)
        scat_start(i0 + 1, rows1_v, semsc1)

        @pl.when(i0 + 2 < NCH)
        def _():
            idx_wait(i0 + 2, 0, semi0)
            pltpu.async_copy(g_hbm.at[idx_row_v.at[0]], rows0_v, sem0)

        @pl.when(i0 + 3 < NCH)
        def _():
            idx_load(i0 + 3, 1, semi1)
        return 0

    lax.fori_loop(0, NCH // 2, cbody, 0, unroll=False)
    scat_wait(NCH - 1, rows1_v, semsc1)
    plsc.subcore_barrier()

    obase = c * N_ACC + row0

    def fbody(i, _):
        pltpu.sync_copy(acc_sh.at[pl.ds(row0 + i * 16, 16)], tmp_v)
        pltpu.sync_copy(tmp_v, out_hbm.at[pl.ds(obase + i * 16, 16)])
        return 0

    lax.fori_loop(0, RPT // 16, fbody, 0, unroll=False)


_spmv_call = pl.kernel(
    _spmv_body,
    out_type=jax.ShapeDtypeStruct((NC * N_ACC, F), jnp.float32),
    mesh=_sc_mesh,
    scratch_types=[
        pltpu.VMEM((2, CHUNK), jnp.int32),
        pltpu.VMEM((4, CHUNK), jnp.int32),
        pltpu.VMEM((CHUNK, F), jnp.float32),
        pltpu.VMEM((CHUNK, F), jnp.float32),
        pltpu.VMEM((16, F), jnp.float32),
        pltpu.VMEM_SHARED((N_ACC, F), jnp.float32),
        pltpu.SemaphoreType.DMA,
        pltpu.SemaphoreType.DMA,
        pltpu.SemaphoreType.DMA,
        pltpu.SemaphoreType.DMA,
        pltpu.SemaphoreType.DMA,
        pltpu.SemaphoreType.DMA,
    ],
)


def _scale(d0_ref, d1_ref):
    d = d0_ref[:, 0:1] + d1_ref[:, 0:1]
    return jnp.where(d > 0.0, lax.rsqrt(jnp.maximum(d, 1e-30)), 0.0)


def _in_body(x_ref, w_ref, b_ref, d0_ref, d1_ref, h_ref, g_ref, cs_ref):
    h = jnp.maximum(jnp.dot(x_ref[...], w_ref[...],
                            preferred_element_type=jnp.float32) + b_ref[...], 0.0)
    h_ref[...] = h
    g_ref[...] = h * _scale(d0_ref, d1_ref)

    @pl.when(pl.program_id(0) == 0)
    def _():
        cs_ref[...] = jnp.zeros_like(cs_ref)

    cs_ref[...] += jnp.sum(h, axis=0, keepdims=True)


_in_call = pl.pallas_call(
    _in_body,
    grid=(GRID,),
    in_specs=[
        pl.BlockSpec((BN, F), lambda i: (i, 0)),
        pl.BlockSpec((F, F), lambda i: (0, 0)),
        pl.BlockSpec((1, F), lambda i: (0, 0)),
        pl.BlockSpec((BN, F), lambda i: (i, 0)),
        pl.BlockSpec((BN, F), lambda i: (i, 0)),
    ],
    out_specs=[
        pl.BlockSpec((BN, F), lambda i: (i, 0)),
        pl.BlockSpec((BN, F), lambda i: (i, 0)),
        pl.BlockSpec((1, F), lambda i: (0, 0)),
    ],
    out_shape=[
        jax.ShapeDtypeStruct((N, F), jnp.float32),
        jax.ShapeDtypeStruct((N, F), jnp.float32),
        jax.ShapeDtypeStruct((1, F), jnp.float32),
    ],
)


def _layer_body(is_last, h_ref, a0_ref, a1_ref, d0_ref, d1_ref, cs_ref,
                wl_ref, fcw_ref, fcb_ref, w1_ref, b1_ref, w2_ref, b2_ref,
                alpha_ref, conv_ref, wo_ref, bo_ref, *outs):
    s = _scale(d0_ref, d1_ref)
    hi = (a0_ref[...] + a1_ref[...]) * s
    h = h_ref[...]

    gp = cs_ref[...] * (1.0 / N)
    ge = jnp.dot(jnp.maximum(jnp.dot(gp, w1_ref[...],
                                     preferred_element_type=jnp.float32)
                             + b1_ref[...], 0.0),
                 w2_ref[...], preferred_element_type=jnp.float32) + b2_ref[...]
    wgt = jax.nn.sigmoid(alpha_ref[0, 0])
    m = jnp.dot(wl_ref[...], fcw_ref[...], preferred_element_type=jnp.float32)
    logits = (wgt * jnp.dot(hi, m, preferred_element_type=jnp.float32)
              + ((1.0 - wgt) * jnp.dot(ge, fcw_ref[...],
                                       preferred_element_type=jnp.float32)
                 + fcb_ref[...]))
    e = jax.nn.softmax(logits, axis=-1)

    hcat = jnp.concatenate([hi, h], axis=1)
    outs_all = jnp.dot(hcat, conv_ref[...], preferred_element_type=jnp.float32)
    out = h
    for k in range(K):
        out = out + e[:, k:k + 1] * outs_all[:, k * F:(k + 1) * F]
    hn = jnp.maximum(out, 0.0)

    if is_last:
        outs[0][...] = jnp.dot(hn, wo_ref[...],
                               preferred_element_type=jnp.float32) + bo_ref[...]
    else:
        outs[0][...] = hn
        outs[1][...] = hn * s

        @pl.when(pl.program_id(0) == 0)
        def _():
            outs[2][...] = jnp.zeros_like(outs[2])

        outs[2][...] += jnp.sum(hn, axis=0, keepdims=True)


def _make_layer_call(is_last):
    full = lambda i: (0, 0)
    in_specs = [
        pl.BlockSpec((BN, F), lambda i: (i, 0)),
        pl.BlockSpec((BN, F), lambda i: (i, 0)),
        pl.BlockSpec((BN, F), lambda i: (i, 0)),
        pl.BlockSpec((BN, F), lambda i: (i, 0)),
        pl.BlockSpec((BN, F), lambda i: (i, 0)),
        pl.BlockSpec((1, F), full),
        pl.BlockSpec((F, F), full),
        pl.BlockSpec((F, K), full),
        pl.BlockSpec((1, K), full),
        pl.BlockSpec((F, F), full),
        pl.BlockSpec((1, F), full),
        pl.BlockSpec((F, F), full),
        pl.BlockSpec((1, F), full),
        pl.BlockSpec((1, 1), full),
        pl.BlockSpec((2 * F, K * F), full),
        pl.BlockSpec((F, F), full),
        pl.BlockSpec((1, F), full),
    ]
    if is_last:
        out_specs = [pl.BlockSpec((BN, F), lambda i: (i, 0))]
        out_shape = [jax.ShapeDtypeStruct((N, F), jnp.float32)]
    else:
        out_specs = [
            pl.BlockSpec((BN, F), lambda i: (i, 0)),
            pl.BlockSpec((BN, F), lambda i: (i, 0)),
            pl.BlockSpec((1, F), full),
        ]
        out_shape = [
            jax.ShapeDtypeStruct((N, F), jnp.float32),
            jax.ShapeDtypeStruct((N, F), jnp.float32),
            jax.ShapeDtypeStruct((1, F), jnp.float32),
        ]
    return pl.pallas_call(
        functools.partial(_layer_body, is_last),
        grid=(GRID,),
        in_specs=in_specs,
        out_specs=out_specs,
        out_shape=out_shape,
    )


_layer_call = _make_layer_call(False)
_last_call = _make_layer_call(True)


def kernel(x, edge_index, W_in, b_in, conv_w, env_Wlocal, env_mlp_w1, env_mlp_b1,
           env_mlp_w2, env_mlp_b2, env_alpha, env_fc_w, env_fc_b, W_out, b_out):
    row = edge_index[0].astype(jnp.int32)
    col = edge_index[1].astype(jnp.int32)
    pad = E_PAD - E
    ar = jnp.arange(pad, dtype=jnp.int32)
    row_p = jnp.concatenate([row, (ar * 37) % N])
    col_p = jnp.concatenate([col, N + (ar % (N_ACC - N))])

    row_2d = row_p.reshape(E_PAD // CHUNK, CHUNK)
    col_2d = col_p.reshape(E_PAD // CHUNK, CHUNK)

    deg = _deg_call(col_p)
    d0 = deg[:N]
    d1 = deg[N_ACC:N_ACC + N]

    b_in2 = b_in.reshape(1, F)
    b_out2 = b_out.reshape(1, F)

    h, g, cs = _in_call(x, W_in, b_in2, d0, d1)

    for l in range(L):
        acc = _spmv_call(row_2d, col_2d, g)
        a0 = acc[:N]
        a1 = acc[N_ACC:N_ACC + N]
        conv_r = conv_w[l].transpose(1, 0, 2).reshape(2 * F, K * F)
        args = (h, a0, a1, d0, d1, cs,
                env_Wlocal[l], env_fc_w[l], env_fc_b[l].reshape(1, K),
                env_mlp_w1[l], env_mlp_b1[l].reshape(1, F),
                env_mlp_w2[l], env_mlp_b2[l].reshape(1, F),
                env_alpha[l].reshape(1, 1), conv_r, W_out, b_out2)
        if l == L - 1:
            out = _last_call(*args)[0]
        else:
            h, g, cs = _layer_call(*args)
    return out


# speedup vs baseline: 21.8519x; 1.2714x over previous
import functools

import jax
import jax.numpy as jnp
from jax import lax
from jax.experimental import pallas as pl
from jax.experimental.pallas import tpu as pltpu
from jax.experimental.pallas import tpu_sc as plsc

N = 10000
E = 320000
F = 128
K = 3
L = 2

NC = 2
NS = 16
NW = NC * NS
CHUNK = 128
EPT = 10240
E_PAD = NW * EPT
N_ACC = 10240
RPT = N_ACC // NS
DW = 16

BN = 1000
GRID = N // BN

_sc_mesh = plsc.VectorSubcoreMesh(core_axis_name="c", subcore_axis_name="s")


DEG_R = 128


N_FLAT = DEG_R * F
NRED = N_FLAT // NS


def _deg_body(col_hbm, out_hbm, cidx_v, cnt_v, acc16_v, res_v, stage_sh, semi):
    c = lax.axis_index("c")
    s = lax.axis_index("s")
    wid = c * NS + s
    cp = pltpu.async_copy(col_hbm.at[pl.ds(wid * EPT, EPT)], cidx_v, semi)

    z16 = jnp.zeros((16,), jnp.float32)

    def zcnt(i, _):
        cnt_v[pl.ds(i * 16, 16)] = z16
        return 0

    lax.fori_loop(0, N_FLAT // 16, zcnt, 0, unroll=False)
    cp.wait()

    ones16 = jnp.ones((16,), jnp.float32)

    def cnt_body(j, _):
        for k in range(CHUNK // 16):
            idx = cidx_v[pl.ds(j * CHUNK + k * 16, 16)]
            plsc.addupdate_scatter(cnt_v, [idx], ones16)
        return 0

    lax.fori_loop(0, EPT // CHUNK, cnt_body, 0, unroll=False)

    pltpu.sync_copy(cnt_v, stage_sh.at[s])
    plsc.subcore_barrier()
    for t in range(NS):
        pltpu.sync_copy(stage_sh.at[t, pl.ds(s * NRED, NRED)], acc16_v.at[t])

    def red_body(w, _):
        acc = acc16_v[0, pl.ds(w * 16, 16)]
        for t in range(1, NS):
            acc = acc + acc16_v[t, pl.ds(w * 16, 16)]
        res_v[pl.ds(w * 16, 16)] = acc
        return 0

    lax.fori_loop(0, NRED // 16, red_body, 0, unroll=False)
    pltpu.sync_copy(res_v, out_hbm.at[pl.ds(wid * NRED, NRED)])


_deg_call = pl.kernel(
    _deg_body,
    out_type=jax.ShapeDtypeStruct((NC * N_FLAT,), jnp.float32),
    mesh=_sc_mesh,
    compiler_params=pltpu.CompilerParams(needs_layout_passes=False),
    scratch_types=[
        pltpu.VMEM((EPT,), jnp.int32),
        pltpu.VMEM((N_FLAT,), jnp.float32),
        pltpu.VMEM((NS, NRED), jnp.float32),
        pltpu.VMEM((NRED,), jnp.float32),
        pltpu.VMEM_SHARED((NS, N_FLAT), jnp.float32),
        pltpu.SemaphoreType.DMA,
    ],
)


NCH = EPT // CHUNK


def _spmv_body(row_hbm, col_hbm, g_hbm, out_hbm,
               idx_row_v, idx_col_v, rows0_v, rows1_v, tmp_v, acc_sh,
               sem0, sem1, semi0, semi1, semsc0, semsc1):
    c = lax.axis_index("c")
    s = lax.axis_index("s")
    wid = c * NS + s
    for r in range(16):
        for j in range(F // 16):
            tmp_v[r, pl.ds(j * 16, 16)] = jnp.zeros((16,), jnp.float32)
    row0 = s * RPT

    gc0 = wid * NCH

    def zbody(i, _):
        pltpu.sync_copy(tmp_v, acc_sh.at[pl.ds(row0 + i * 16, 16)])
        return 0

    lax.fori_loop(0, RPT // 16, zbody, 0, unroll=False)

    def idx_load(i, rb, sem):
        pltpu.async_copy(row_hbm.at[pl.ds(gc0 + i, 1)],
                         idx_row_v.at[pl.ds(rb, 1)], sem)
        pltpu.async_copy(col_hbm.at[pl.ds(gc0 + i, 1)],
                         idx_col_v.at[pl.ds((gc0 + i) % 4, 1)], sem)

    def idx_wait(i, rb, sem):
        pltpu.make_async_copy(row_hbm.at[pl.ds(gc0 + i, 1)],
                              idx_row_v.at[pl.ds(rb, 1)], sem).wait()
        pltpu.make_async_copy(col_hbm.at[pl.ds(gc0 + i, 1)],
                              idx_col_v.at[pl.ds((gc0 + i) % 4, 1)], sem).wait()

    def scat_start(i, rows, sem):
        pltpu.async_copy(rows, acc_sh.at[idx_col_v.at[(gc0 + i) % 4]], sem,
                         add=True)

    def scat_wait(i, rows, sem):
        pltpu.make_async_copy(rows, acc_sh.at[idx_col_v.at[(gc0 + i) % 4]],
                              sem).wait()

    pltpu.sync_copy(row_hbm.at[pl.ds(gc0, 1)], idx_row_v.at[pl.ds(0, 1)])
    pltpu.sync_copy(col_hbm.at[pl.ds(gc0, 1)],
                    idx_col_v.at[pl.ds(gc0 % 4, 1)])
    plsc.subcore_barrier()
    pltpu.async_copy(g_hbm.at[idx_row_v.at[0]], rows0_v, sem0)
    idx_load(1, 1, semi1)

    def cbody(j, _):
        i0 = 2 * j
        idx_wait(i0 + 1, 1, semi1)

        @pl.when(j > 0)
        def _():
            scat_wait(i0 - 1, rows1_v, semsc1)
        pltpu.async_copy(g_hbm.at[idx_row_v.at[1]], rows1_v, sem1)

        pltpu.make_async_copy(g_hbm.at[idx_row_v.at[0]], rows0_v, sem0).wait()
        scat_start(i0, rows0_v, semsc0)

        @pl.when(i0 + 2 < NCH)
        def _():
            idx_load(i0 + 2, 0, semi0)

        pltpu.make_async_copy(g_hbm.at[idx_row_v.at[1]], rows1_v, sem1).wait()
        scat_wait(i0, rows0_v, semsc0)
        scat_start(i0 + 1, rows1_v, semsc1)

        @pl.when(i0 + 2 < NCH)
        def _():
            idx_wait(i0 + 2, 0, semi0)
            pltpu.async_copy(g_hbm.at[idx_row_v.at[0]], rows0_v, sem0)

        @pl.when(i0 + 3 < NCH)
        def _():
            idx_load(i0 + 3, 1, semi1)
        return 0

    lax.fori_loop(0, NCH // 2, cbody, 0, unroll=False)
    scat_wait(NCH - 1, rows1_v, semsc1)
    plsc.subcore_barrier()

    obase = c * N_ACC + row0

    def fbody(i, _):
        pltpu.sync_copy(acc_sh.at[pl.ds(row0 + i * 16, 16)], tmp_v)
        pltpu.sync_copy(tmp_v, out_hbm.at[pl.ds(obase + i * 16, 16)])
        return 0

    lax.fori_loop(0, RPT // 16, fbody, 0, unroll=False)


_spmv_call = pl.kernel(
    _spmv_body,
    out_type=jax.ShapeDtypeStruct((NC * N_ACC, F), jnp.float32),
    mesh=_sc_mesh,
    scratch_types=[
        pltpu.VMEM((2, CHUNK), jnp.int32),
        pltpu.VMEM((4, CHUNK), jnp.int32),
        pltpu.VMEM((CHUNK, F), jnp.float32),
        pltpu.VMEM((CHUNK, F), jnp.float32),
        pltpu.VMEM((16, F), jnp.float32),
        pltpu.VMEM_SHARED((N_ACC, F), jnp.float32),
        pltpu.SemaphoreType.DMA,
        pltpu.SemaphoreType.DMA,
        pltpu.SemaphoreType.DMA,
        pltpu.SemaphoreType.DMA,
        pltpu.SemaphoreType.DMA,
        pltpu.SemaphoreType.DMA,
    ],
)


def _scale(d_ref):
    d = d_ref[...]
    return jnp.where(d > 0.0, lax.rsqrt(jnp.maximum(d, 1e-30)), 0.0)


def _in_body(x_ref, w_ref, b_ref, d_ref, h_ref, g_ref, cs_ref):
    h = jnp.maximum(jnp.dot(x_ref[...], w_ref[...],
                            preferred_element_type=jnp.float32) + b_ref[...], 0.0)
    h_ref[...] = h
    g_ref[...] = h * _scale(d_ref)

    @pl.when(pl.program_id(0) == 0)
    def _():
        cs_ref[...] = jnp.zeros_like(cs_ref)

    cs_ref[...] += jnp.sum(h, axis=0, keepdims=True)


_in_call = pl.pallas_call(
    _in_body,
    grid=(GRID,),
    in_specs=[
        pl.BlockSpec((BN, F), lambda i: (i, 0)),
        pl.BlockSpec((F, F), lambda i: (0, 0)),
        pl.BlockSpec((1, F), lambda i: (0, 0)),
        pl.BlockSpec((BN, 1), lambda i: (i, 0)),
    ],
    out_specs=[
        pl.BlockSpec((BN, F), lambda i: (i, 0)),
        pl.BlockSpec((BN, F), lambda i: (i, 0)),
        pl.BlockSpec((1, F), lambda i: (0, 0)),
    ],
    out_shape=[
        jax.ShapeDtypeStruct((N, F), jnp.float32),
        jax.ShapeDtypeStruct((N, F), jnp.float32),
        jax.ShapeDtypeStruct((1, F), jnp.float32),
    ],
)


def _layer_body(is_last, h_ref, a0_ref, a1_ref, d_ref, cs_ref,
                wl_ref, fcw_ref, fcb_ref, w1_ref, b1_ref, w2_ref, b2_ref,
                alpha_ref, conv_ref, wo_ref, bo_ref, *outs):
    s = _scale(d_ref)
    hi = (a0_ref[...] + a1_ref[...]) * s
    h = h_ref[...]

    gp = cs_ref[...] * (1.0 / N)
    ge = jnp.dot(jnp.maximum(jnp.dot(gp, w1_ref[...],
                                     preferred_element_type=jnp.float32)
                             + b1_ref[...], 0.0),
                 w2_ref[...], preferred_element_type=jnp.float32) + b2_ref[...]
    wgt = jax.nn.sigmoid(alpha_ref[0, 0])
    m = jnp.dot(wl_ref[...], fcw_ref[...], preferred_element_type=jnp.float32)
    logits = (wgt * jnp.dot(hi, m, preferred_element_type=jnp.float32)
              + ((1.0 - wgt) * jnp.dot(ge, fcw_ref[...],
                                       preferred_element_type=jnp.float32)
                 + fcb_ref[...]))
    e = jax.nn.softmax(logits, axis=-1)

    hcat = jnp.concatenate([hi, h], axis=1)
    outs_all = jnp.dot(hcat, conv_ref[...], preferred_element_type=jnp.float32)
    out = h
    for k in range(K):
        out = out + e[:, k:k + 1] * outs_all[:, k * F:(k + 1) * F]
    hn = jnp.maximum(out, 0.0)

    if is_last:
        outs[0][...] = jnp.dot(hn, wo_ref[...],
                               preferred_element_type=jnp.float32) + bo_ref[...]
    else:
        outs[0][...] = hn
        outs[1][...] = hn * s

        @pl.when(pl.program_id(0) == 0)
        def _():
            outs[2][...] = jnp.zeros_like(outs[2])

        outs[2][...] += jnp.sum(hn, axis=0, keepdims=True)


def _make_layer_call(is_last):
    full = lambda i: (0, 0)
    in_specs = [
        pl.BlockSpec((BN, F), lambda i: (i, 0)),
        pl.BlockSpec((BN, F), lambda i: (i, 0)),
        pl.BlockSpec((BN, F), lambda i: (i, 0)),
        pl.BlockSpec((BN, 1), lambda i: (i, 0)),
        pl.BlockSpec((1, F), full),
        pl.BlockSpec((F, F), full),
        pl.BlockSpec((F, K), full),
        pl.BlockSpec((1, K), full),
        pl.BlockSpec((F, F), full),
        pl.BlockSpec((1, F), full),
        pl.BlockSpec((F, F), full),
        pl.BlockSpec((1, F), full),
        pl.BlockSpec((1, 1), full),
        pl.BlockSpec((2 * F, K * F), full),
        pl.BlockSpec((F, F), full),
        pl.BlockSpec((1, F), full),
    ]
    if is_last:
        out_specs = [pl.BlockSpec((BN, F), lambda i: (i, 0))]
        out_shape = [jax.ShapeDtypeStruct((N, F), jnp.float32)]
    else:
        out_specs = [
            pl.BlockSpec((BN, F), lambda i: (i, 0)),
            pl.BlockSpec((BN, F), lambda i: (i, 0)),
            pl.BlockSpec((1, F), full),
        ]
        out_shape = [
            jax.ShapeDtypeStruct((N, F), jnp.float32),
            jax.ShapeDtypeStruct((N, F), jnp.float32),
            jax.ShapeDtypeStruct((1, F), jnp.float32),
        ]
    return pl.pallas_call(
        functools.partial(_layer_body, is_last),
        grid=(GRID,),
        in_specs=in_specs,
        out_specs=out_specs,
        out_shape=out_shape,
    )


_layer_call = _make_layer_call(False)
_last_call = _make_layer_call(True)


def kernel(x, edge_index, W_in, b_in, conv_w, env_Wlocal, env_mlp_w1, env_mlp_b1,
           env_mlp_w2, env_mlp_b2, env_alpha, env_fc_w, env_fc_b, W_out, b_out):
    row = edge_index[0].astype(jnp.int32)
    col = edge_index[1].astype(jnp.int32)
    pad = E_PAD - E
    ar = jnp.arange(pad, dtype=jnp.int32)
    row_p = jnp.concatenate([row, (ar * 37) % N])
    col_p = jnp.concatenate([col, N + (ar % (N_ACC - N))])

    row_2d = row_p.reshape(E_PAD // CHUNK, CHUNK)
    col_2d = col_p.reshape(E_PAD // CHUNK, CHUNK)

    deg = _deg_call(col_p)
    d = (deg[:N_FLAT] + deg[N_FLAT:])[:N].reshape(N, 1)

    b_in2 = b_in.reshape(1, F)
    b_out2 = b_out.reshape(1, F)

    h, g, cs = _in_call(x, W_in, b_in2, d)

    for l in range(L):
        acc = _spmv_call(row_2d, col_2d, g)
        a0 = acc[:N]
        a1 = acc[N_ACC:N_ACC + N]
        conv_r = conv_w[l].transpose(1, 0, 2).reshape(2 * F, K * F)
        args = (h, a0, a1, d, cs,
                env_Wlocal[l], env_fc_w[l], env_fc_b[l].reshape(1, K),
                env_mlp_w1[l], env_mlp_b1[l].reshape(1, F),
                env_mlp_w2[l], env_mlp_b2[l].reshape(1, F),
                env_alpha[l].reshape(1, 1), conv_r, W_out, b_out2)
        if l == L - 1:
            out = _last_call(*args)[0]
        else:
            h, g, cs = _layer_call(*args)
    return out


# speedup vs baseline: 23.1363x; 1.0588x over previous
import functools

import jax
import jax.numpy as jnp
from jax import lax
from jax.experimental import pallas as pl
from jax.experimental.pallas import tpu as pltpu
from jax.experimental.pallas import tpu_sc as plsc

N = 10000
E = 320000
F = 128
K = 3
L = 2

NC = 2
NS = 16
NW = NC * NS
CHUNK = 128
NCHT = E // CHUNK
NCH0 = NCHT // NW
XTRA = NCHT - NW * NCH0
EPT_DEG = E // NW
N_ACC = N
RPB = 624
FR = 16
DW = 16

BN = 1000
GRID = N // BN

_sc_mesh = plsc.VectorSubcoreMesh(core_axis_name="c", subcore_axis_name="s")


DEG_R = 128


N_FLAT = DEG_R * F
NRED = N_FLAT // NS


def _deg_body(col_hbm, out_hbm, cidx_v, cnt_v, acc16_v, res_v, stage_sh, semi):
    c = lax.axis_index("c")
    s = lax.axis_index("s")
    wid = c * NS + s
    cp = pltpu.async_copy(col_hbm.at[pl.ds(wid * EPT_DEG, EPT_DEG)], cidx_v,
                          semi)

    z16 = jnp.zeros((16,), jnp.float32)

    def zcnt(i, _):
        cnt_v[pl.ds(i * 16, 16)] = z16
        return 0

    lax.fori_loop(0, N_FLAT // 16, zcnt, 0, unroll=False)
    cp.wait()

    ones16 = jnp.ones((16,), jnp.float32)

    def cnt_body(j, _):
        for k in range(5):
            idx = cidx_v[pl.ds(j * 80 + k * 16, 16)]
            plsc.addupdate_scatter(cnt_v, [idx], ones16)
        return 0

    lax.fori_loop(0, EPT_DEG // 80, cnt_body, 0, unroll=False)

    pltpu.sync_copy(cnt_v, stage_sh.at[s])
    plsc.subcore_barrier()
    for t in range(NS):
        pltpu.sync_copy(stage_sh.at[t, pl.ds(s * NRED, NRED)], acc16_v.at[t])

    def red_body(w, _):
        acc = acc16_v[0, pl.ds(w * 16, 16)]
        for t in range(1, NS):
            acc = acc + acc16_v[t, pl.ds(w * 16, 16)]
        res_v[pl.ds(w * 16, 16)] = acc
        return 0

    lax.fori_loop(0, NRED // 16, red_body, 0, unroll=False)
    pltpu.sync_copy(res_v, out_hbm.at[pl.ds(wid * NRED, NRED)])


_deg_call = pl.kernel(
    _deg_body,
    out_type=jax.ShapeDtypeStruct((NC * N_FLAT,), jnp.float32),
    mesh=_sc_mesh,
    compiler_params=pltpu.CompilerParams(needs_layout_passes=False),
    scratch_types=[
        pltpu.VMEM((EPT_DEG,), jnp.int32),
        pltpu.VMEM((N_FLAT,), jnp.float32),
        pltpu.VMEM((NS, NRED), jnp.float32),
        pltpu.VMEM((NRED,), jnp.float32),
        pltpu.VMEM_SHARED((NS, N_FLAT), jnp.float32),
        pltpu.SemaphoreType.DMA,
    ],
)


def _spmv_body(row_hbm, col_hbm, g_hbm, out_hbm,
               idx_row_v, idx_col_v, rows0_v, rows1_v, tmp_v, acc_sh,
               sem0, sem1, semi0, semi1, semsc0, semsc1):
    c = lax.axis_index("c")
    s = lax.axis_index("s")
    wid = c * NS + s
    for r in range(FR):
        for j in range(F // 16):
            tmp_v[r, pl.ds(j * 16, 16)] = jnp.zeros((16,), jnp.float32)
    row0 = s * RPB
    nfl = (RPB + jnp.where(s == NS - 1, N_ACC - NS * RPB, 0)) // FR

    nch = NCH0 + jnp.where(wid < XTRA, 1, 0)
    gc0 = wid * NCH0 + jnp.minimum(wid, XTRA)

    def zbody(i, _):
        pltpu.sync_copy(tmp_v, acc_sh.at[pl.ds(row0 + i * FR, FR)])
        return 0

    lax.fori_loop(0, nfl, zbody, 0, unroll=False)

    def idx_load(i, rb, sem):
        pltpu.async_copy(row_hbm.at[pl.ds((gc0 + i) * CHUNK, CHUNK)],
                         idx_row_v.at[rb], sem)
        pltpu.async_copy(col_hbm.at[pl.ds((gc0 + i) * CHUNK, CHUNK)],
                         idx_col_v.at[(gc0 + i) % 4], sem)

    def idx_wait(i, rb, sem):
        pltpu.make_async_copy(row_hbm.at[pl.ds((gc0 + i) * CHUNK, CHUNK)],
                              idx_row_v.at[rb], sem).wait()
        pltpu.make_async_copy(col_hbm.at[pl.ds((gc0 + i) * CHUNK, CHUNK)],
                              idx_col_v.at[(gc0 + i) % 4], sem).wait()

    def scat_start(i, rows, sem):
        pltpu.async_copy(rows, acc_sh.at[idx_col_v.at[(gc0 + i) % 4]], sem,
                         add=True)

    def scat_wait(i, rows, sem):
        pltpu.make_async_copy(rows, acc_sh.at[idx_col_v.at[(gc0 + i) % 4]],
                              sem).wait()

    pltpu.sync_copy(row_hbm.at[pl.ds(gc0 * CHUNK, CHUNK)], idx_row_v.at[0])
    pltpu.sync_copy(col_hbm.at[pl.ds(gc0 * CHUNK, CHUNK)],
                    idx_col_v.at[gc0 % 4])
    plsc.subcore_barrier()
    pltpu.async_copy(g_hbm.at[idx_row_v.at[0]], rows0_v, sem0)
    idx_load(1, 1, semi1)

    def cbody(j, _):
        i0 = 2 * j
        idx_wait(i0 + 1, 1, semi1)

        @pl.when(j > 0)
        def _():
            scat_wait(i0 - 1, rows1_v, semsc1)
        pltpu.async_copy(g_hbm.at[idx_row_v.at[1]], rows1_v, sem1)

        pltpu.make_async_copy(g_hbm.at[idx_row_v.at[0]], rows0_v, sem0).wait()
        scat_start(i0, rows0_v, semsc0)

        @pl.when(i0 + 2 < nch)
        def _():
            idx_load(i0 + 2, 0, semi0)

        pltpu.make_async_copy(g_hbm.at[idx_row_v.at[1]], rows1_v, sem1).wait()
        scat_wait(i0, rows0_v, semsc0)
        scat_start(i0 + 1, rows1_v, semsc1)

        @pl.when(i0 + 2 < nch)
        def _():
            idx_wait(i0 + 2, 0, semi0)
            pltpu.async_copy(g_hbm.at[idx_row_v.at[0]], rows0_v, sem0)

        @pl.when(i0 + 3 < nch)
        def _():
            idx_load(i0 + 3, 1, semi1)
        return 0

    lax.fori_loop(0, NCH0 // 2, cbody, 0, unroll=False)
    scat_wait(2 * (NCH0 // 2) - 1, rows1_v, semsc1)

    @pl.when(nch > NCH0)
    def _():
        i = NCH0
        pltpu.make_async_copy(g_hbm.at[idx_row_v.at[0]], rows0_v, sem0).wait()
        pltpu.sync_copy(rows0_v, acc_sh.at[idx_col_v.at[(gc0 + i) % 4]],
                        add=True)

    plsc.subcore_barrier()

    obase = c * N_ACC + row0

    def fbody(i, _):
        pltpu.sync_copy(acc_sh.at[pl.ds(row0 + i * FR, FR)], tmp_v)
        pltpu.sync_copy(tmp_v, out_hbm.at[pl.ds(obase + i * FR, FR)])
        return 0

    lax.fori_loop(0, nfl, fbody, 0, unroll=False)


_spmv_call = pl.kernel(
    _spmv_body,
    out_type=jax.ShapeDtypeStruct((NC * N_ACC, F), jnp.float32),
    mesh=_sc_mesh,
    scratch_types=[
        pltpu.VMEM((2, CHUNK), jnp.int32),
        pltpu.VMEM((4, CHUNK), jnp.int32),
        pltpu.VMEM((CHUNK, F), jnp.float32),
        pltpu.VMEM((CHUNK, F), jnp.float32),
        pltpu.VMEM((FR, F), jnp.float32),
        pltpu.VMEM_SHARED((N_ACC, F), jnp.float32),
        pltpu.SemaphoreType.DMA,
        pltpu.SemaphoreType.DMA,
        pltpu.SemaphoreType.DMA,
        pltpu.SemaphoreType.DMA,
        pltpu.SemaphoreType.DMA,
        pltpu.SemaphoreType.DMA,
    ],
)


def _scale(d_ref):
    d = d_ref[...]
    return jnp.where(d > 0.0, lax.rsqrt(jnp.maximum(d, 1e-30)), 0.0)


def _h_body(x_ref, w_ref, b_ref, h_ref, cs_ref):
    h = jnp.maximum(jnp.dot(x_ref[...], w_ref[...],
                            preferred_element_type=jnp.float32) + b_ref[...], 0.0)
    h_ref[...] = h

    @pl.when(pl.program_id(0) == 0)
    def _():
        cs_ref[...] = jnp.zeros_like(cs_ref)

    cs_ref[...] += jnp.sum(h, axis=0, keepdims=True)


_h_call = pl.pallas_call(
    _h_body,
    grid=(GRID,),
    in_specs=[
        pl.BlockSpec((BN, F), lambda i: (i, 0)),
        pl.BlockSpec((F, F), lambda i: (0, 0)),
        pl.BlockSpec((1, F), lambda i: (0, 0)),
    ],
    out_specs=[
        pl.BlockSpec((BN, F), lambda i: (i, 0)),
        pl.BlockSpec((1, F), lambda i: (0, 0)),
    ],
    out_shape=[
        jax.ShapeDtypeStruct((N, F), jnp.float32),
        jax.ShapeDtypeStruct((1, F), jnp.float32),
    ],
)


def _g_body(h_ref, d_ref, g_ref):
    g_ref[...] = h_ref[...] * _scale(d_ref)


_g_call = pl.pallas_call(
    _g_body,
    grid=(GRID,),
    in_specs=[
        pl.BlockSpec((BN, F), lambda i: (i, 0)),
        pl.BlockSpec((BN, 1), lambda i: (i, 0)),
    ],
    out_specs=pl.BlockSpec((BN, F), lambda i: (i, 0)),
    out_shape=jax.ShapeDtypeStruct((N, F), jnp.float32),
)


def _layer_body(is_last, h_ref, a0_ref, a1_ref, d_ref, cs_ref,
                wl_ref, fcw_ref, fcb_ref, w1_ref, b1_ref, w2_ref, b2_ref,
                alpha_ref, conv_ref, wo_ref, bo_ref, *outs):
    s = _scale(d_ref)
    hi = (a0_ref[...] + a1_ref[...]) * s
    h = h_ref[...]

    gp = cs_ref[...] * (1.0 / N)
    ge = jnp.dot(jnp.maximum(jnp.dot(gp, w1_ref[...],
                                     preferred_element_type=jnp.float32)
                             + b1_ref[...], 0.0),
                 w2_ref[...], preferred_element_type=jnp.float32) + b2_ref[...]
    wgt = jax.nn.sigmoid(alpha_ref[0, 0])
    m = jnp.dot(wl_ref[...], fcw_ref[...], preferred_element_type=jnp.float32)
    logits = (wgt * jnp.dot(hi, m, preferred_element_type=jnp.float32)
              + ((1.0 - wgt) * jnp.dot(ge, fcw_ref[...],
                                       preferred_element_type=jnp.float32)
                 + fcb_ref[...]))
    e = jax.nn.softmax(logits, axis=-1)

    hcat = jnp.concatenate([hi, h], axis=1)
    outs_all = jnp.dot(hcat, conv_ref[...], preferred_element_type=jnp.float32)
    out = h
    for k in range(K):
        out = out + e[:, k:k + 1] * outs_all[:, k * F:(k + 1) * F]
    hn = jnp.maximum(out, 0.0)

    if is_last:
        outs[0][...] = jnp.dot(hn, wo_ref[...],
                               preferred_element_type=jnp.float32) + bo_ref[...]
    else:
        outs[0][...] = hn
        outs[1][...] = hn * s

        @pl.when(pl.program_id(0) == 0)
        def _():
            outs[2][...] = jnp.zeros_like(outs[2])

        outs[2][...] += jnp.sum(hn, axis=0, keepdims=True)


def _make_layer_call(is_last):
    full = lambda i: (0, 0)
    in_specs = [
        pl.BlockSpec((BN, F), lambda i: (i, 0)),
        pl.BlockSpec((BN, F), lambda i: (i, 0)),
        pl.BlockSpec((BN, F), lambda i: (GRID + i, 0)),
        pl.BlockSpec((BN, 1), lambda i: (i, 0)),
        pl.BlockSpec((1, F), full),
        pl.BlockSpec((F, F), full),
        pl.BlockSpec((F, K), full),
        pl.BlockSpec((1, K), full),
        pl.BlockSpec((F, F), full),
        pl.BlockSpec((1, F), full),
        pl.BlockSpec((F, F), full),
        pl.BlockSpec((1, F), full),
        pl.BlockSpec((1, 1), full),
        pl.BlockSpec((2 * F, K * F), full),
        pl.BlockSpec((F, F), full),
        pl.BlockSpec((1, F), full),
    ]
    if is_last:
        out_specs = [pl.BlockSpec((BN, F), lambda i: (i, 0))]
        out_shape = [jax.ShapeDtypeStruct((N, F), jnp.float32)]
    else:
        out_specs = [
            pl.BlockSpec((BN, F), lambda i: (i, 0)),
            pl.BlockSpec((BN, F), lambda i: (i, 0)),
            pl.BlockSpec((1, F), full),
        ]
        out_shape = [
            jax.ShapeDtypeStruct((N, F), jnp.float32),
            jax.ShapeDtypeStruct((N, F), jnp.float32),
            jax.ShapeDtypeStruct((1, F), jnp.float32),
        ]
    return pl.pallas_call(
        functools.partial(_layer_body, is_last),
        grid=(GRID,),
        in_specs=in_specs,
        out_specs=out_specs,
        out_shape=out_shape,
    )


_layer_call = _make_layer_call(False)
_last_call = _make_layer_call(True)


def kernel(x, edge_index, W_in, b_in, conv_w, env_Wlocal, env_mlp_w1, env_mlp_b1,
           env_mlp_w2, env_mlp_b2, env_alpha, env_fc_w, env_fc_b, W_out, b_out):
    row = edge_index[0].astype(jnp.int32)
    col = edge_index[1].astype(jnp.int32)

    deg = _deg_call(col)
    d = (deg[:N_FLAT] + deg[N_FLAT:])[:N].reshape(N, 1)

    b_in2 = b_in.reshape(1, F)
    b_out2 = b_out.reshape(1, F)

    h, cs = _h_call(x, W_in, b_in2)
    g = _g_call(h, d)

    for l in range(L):
        acc = _spmv_call(row, col, g)
        conv_r = conv_w[l].transpose(1, 0, 2).reshape(2 * F, K * F)
        args = (h, acc, acc, d, cs,
                env_Wlocal[l], env_fc_w[l], env_fc_b[l].reshape(1, K),
                env_mlp_w1[l], env_mlp_b1[l].reshape(1, F),
                env_mlp_w2[l], env_mlp_b2[l].reshape(1, F),
                env_alpha[l].reshape(1, 1), conv_r, W_out, b_out2)
        if l == L - 1:
            out = _last_call(*args)[0]
        else:
            h, g, cs = _layer_call(*args)
    return out


# speedup vs baseline: 25.6693x; 1.1095x over previous
import functools

import jax
import jax.numpy as jnp
from jax import lax
from jax.experimental import pallas as pl
from jax.experimental.pallas import tpu as pltpu
from jax.experimental.pallas import tpu_sc as plsc

N = 10000
E = 320000
F = 128
K = 3
L = 2

NC = 2
NS = 16
NW = NC * NS
CHUNK = 128
NCHT = E // CHUNK
NCH0 = NCHT // NW
XTRA = NCHT - NW * NCH0
EPT_DEG = E // NW
N_ACC = N
RPB = 624
FR = 16
DW = 16

BN = 1000
GRID = N // BN

_sc_mesh = plsc.VectorSubcoreMesh(core_axis_name="c", subcore_axis_name="s")


DEG_R = 128


N_FLAT = DEG_R * F
NRED = N_FLAT // NS


def _deg_body(col_hbm, out_hbm, cidx_v, cnt_v, acc16_v, res_v, stage_sh, semi):
    c = lax.axis_index("c")
    s = lax.axis_index("s")
    wid = c * NS + s
    cp = pltpu.async_copy(col_hbm.at[pl.ds(wid * EPT_DEG, EPT_DEG)], cidx_v,
                          semi)

    z16 = jnp.zeros((16,), jnp.float32)

    def zcnt(i, _):
        cnt_v[pl.ds(i * 16, 16)] = z16
        return 0

    lax.fori_loop(0, N_FLAT // 16, zcnt, 0, unroll=False)
    cp.wait()

    ones16 = jnp.ones((16,), jnp.float32)

    def cnt_body(j, _):
        for k in range(5):
            idx = cidx_v[pl.ds(j * 80 + k * 16, 16)]
            plsc.addupdate_scatter(cnt_v, [idx], ones16)
        return 0

    lax.fori_loop(0, EPT_DEG // 80, cnt_body, 0, unroll=False)

    pltpu.sync_copy(cnt_v, stage_sh.at[s])
    plsc.subcore_barrier()
    for t in range(NS):
        pltpu.sync_copy(stage_sh.at[t, pl.ds(s * NRED, NRED)], acc16_v.at[t])

    def red_body(w, _):
        acc = acc16_v[0, pl.ds(w * 16, 16)]
        for t in range(1, NS):
            acc = acc + acc16_v[t, pl.ds(w * 16, 16)]
        res_v[pl.ds(w * 16, 16)] = acc
        return 0

    lax.fori_loop(0, NRED // 16, red_body, 0, unroll=False)
    pltpu.sync_copy(res_v, out_hbm.at[pl.ds(wid * NRED, NRED)])


_deg_call = pl.kernel(
    _deg_body,
    out_type=jax.ShapeDtypeStruct((NC * N_FLAT,), jnp.float32),
    mesh=_sc_mesh,
    compiler_params=pltpu.CompilerParams(needs_layout_passes=False),
    scratch_types=[
        pltpu.VMEM((EPT_DEG,), jnp.int32),
        pltpu.VMEM((N_FLAT,), jnp.float32),
        pltpu.VMEM((NS, NRED), jnp.float32),
        pltpu.VMEM((NRED,), jnp.float32),
        pltpu.VMEM_SHARED((NS, N_FLAT), jnp.float32),
        pltpu.SemaphoreType.DMA,
    ],
)


def _spmv_body(row_hbm, col_hbm, g_hbm, out_hbm,
               idx_row_v, idx_col_v, rows0_v, rows1_v, rows2_v, acc_sh,
               sg0, sg1, sg2, ss0, ss1, ss2, si0, si1, si2):
    c = lax.axis_index("c")
    s = lax.axis_index("s")
    wid = c * NS + s
    rows = (rows0_v, rows1_v, rows2_v)
    sg = (sg0, sg1, sg2)
    ss = (ss0, ss1, ss2)
    si = (si0, si1, si2)

    for r in range(FR):
        for j in range(F // 16):
            rows0_v[r, pl.ds(j * 16, 16)] = jnp.zeros((16,), jnp.float32)
    row0 = s * RPB
    nfl = (RPB + jnp.where(s == NS - 1, N_ACC - NS * RPB, 0)) // FR

    nch = NCH0 + jnp.where(wid < XTRA, 1, 0)
    gc0 = wid * NCH0 + jnp.minimum(wid, XTRA)

    def zbody(i, _):
        pltpu.sync_copy(rows0_v.at[pl.ds(0, FR)],
                        acc_sh.at[pl.ds(row0 + i * FR, FR)])
        return 0

    lax.fori_loop(0, nfl, zbody, 0, unroll=False)

    def idx_load(i, sem):
        pltpu.async_copy(row_hbm.at[pl.ds((gc0 + i) * CHUNK, CHUNK)],
                         idx_row_v.at[(gc0 + i) % 3], sem)
        pltpu.async_copy(col_hbm.at[pl.ds((gc0 + i) * CHUNK, CHUNK)],
                         idx_col_v.at[(gc0 + i) % 4], sem)

    def idx_wait(i, sem):
        pltpu.make_async_copy(row_hbm.at[pl.ds((gc0 + i) * CHUNK, CHUNK)],
                              idx_row_v.at[(gc0 + i) % 3], sem).wait()
        pltpu.make_async_copy(col_hbm.at[pl.ds((gc0 + i) * CHUNK, CHUNK)],
                              idx_col_v.at[(gc0 + i) % 4], sem).wait()

    def gat_start(i, b):
        pltpu.async_copy(g_hbm.at[idx_row_v.at[(gc0 + i) % 3]], rows[b],
                         sg[b])

    def gat_wait(i, b):
        pltpu.make_async_copy(g_hbm.at[idx_row_v.at[(gc0 + i) % 3]], rows[b],
                              sg[b]).wait()

    def scat_start(i, b):
        pltpu.async_copy(rows[b], acc_sh.at[idx_col_v.at[(gc0 + i) % 4]],
                         ss[b], add=True)

    def scat_wait(i, b):
        pltpu.make_async_copy(rows[b], acc_sh.at[idx_col_v.at[(gc0 + i) % 4]],
                              ss[b]).wait()

    pltpu.sync_copy(row_hbm.at[pl.ds(gc0 * CHUNK, CHUNK)],
                    idx_row_v.at[gc0 % 3])
    pltpu.sync_copy(col_hbm.at[pl.ds(gc0 * CHUNK, CHUNK)],
                    idx_col_v.at[gc0 % 4])
    plsc.subcore_barrier()
    gat_start(0, 0)
    idx_load(1, si[1])
    idx_load(2, si[2])
    idx_wait(1, si[1])
    gat_start(1, 1)

    def cbody(j, _):
        for b in range(3):
            ii = 3 * j + b
            b2 = (b + 2) % 3
            if b == 0:
                @pl.when(j > 0)
                def _():
                    scat_wait(ii - 1, b2)
            else:
                scat_wait(ii - 1, b2)

            @pl.when(ii + 2 < nch)
            def _():
                idx_wait(ii + 2, si[b2])
                gat_start(ii + 2, b2)

            gat_wait(ii, b)
            scat_start(ii, b)

            @pl.when(ii + 3 < nch)
            def _():
                idx_load(ii + 3, si[b])
        return 0

    lax.fori_loop(0, NCH0 // 3, cbody, 0, unroll=False)
    scat_wait(NCH0 - 1, (NCH0 - 1) % 3)

    @pl.when(nch > NCH0)
    def _():
        i = NCH0
        b = i % 3
        gat_wait(i, b)
        pltpu.sync_copy(rows[b], acc_sh.at[idx_col_v.at[(gc0 + i) % 4]],
                        add=True)

    plsc.subcore_barrier()

    obase = c * N_ACC + row0

    def fbody(i, _):
        pltpu.sync_copy(acc_sh.at[pl.ds(row0 + i * FR, FR)],
                        rows0_v.at[pl.ds(0, FR)])
        pltpu.sync_copy(rows0_v.at[pl.ds(0, FR)],
                        out_hbm.at[pl.ds(obase + i * FR, FR)])
        return 0

    lax.fori_loop(0, nfl, fbody, 0, unroll=False)


_spmv_call = pl.kernel(
    _spmv_body,
    out_type=jax.ShapeDtypeStruct((NC * N_ACC, F), jnp.float32),
    mesh=_sc_mesh,
    scratch_types=[
        pltpu.VMEM((3, CHUNK), jnp.int32),
        pltpu.VMEM((4, CHUNK), jnp.int32),
        pltpu.VMEM((CHUNK, F), jnp.float32),
        pltpu.VMEM((CHUNK, F), jnp.float32),
        pltpu.VMEM((CHUNK, F), jnp.float32),
        pltpu.VMEM_SHARED((N_ACC, F), jnp.float32),
        pltpu.SemaphoreType.DMA,
        pltpu.SemaphoreType.DMA,
        pltpu.SemaphoreType.DMA,
        pltpu.SemaphoreType.DMA,
        pltpu.SemaphoreType.DMA,
        pltpu.SemaphoreType.DMA,
        pltpu.SemaphoreType.DMA,
        pltpu.SemaphoreType.DMA,
        pltpu.SemaphoreType.DMA,
    ],
)


def _scale(d_ref):
    d = d_ref[...]
    return jnp.where(d > 0.0, lax.rsqrt(jnp.maximum(d, 1e-30)), 0.0)


def _h_body(x_ref, w_ref, b_ref, h_ref, cs_ref):
    h = jnp.maximum(jnp.dot(x_ref[...], w_ref[...],
                            preferred_element_type=jnp.float32) + b_ref[...], 0.0)
    h_ref[...] = h

    @pl.when(pl.program_id(0) == 0)
    def _():
        cs_ref[...] = jnp.zeros_like(cs_ref)

    cs_ref[...] += jnp.sum(h, axis=0, keepdims=True)


_h_call = pl.pallas_call(
    _h_body,
    grid=(GRID,),
    in_specs=[
        pl.BlockSpec((BN, F), lambda i: (i, 0)),
        pl.BlockSpec((F, F), lambda i: (0, 0)),
        pl.BlockSpec((1, F), lambda i: (0, 0)),
    ],
    out_specs=[
        pl.BlockSpec((BN, F), lambda i: (i, 0)),
        pl.BlockSpec((1, F), lambda i: (0, 0)),
    ],
    out_shape=[
        jax.ShapeDtypeStruct((N, F), jnp.float32),
        jax.ShapeDtypeStruct((1, F), jnp.float32),
    ],
)


def _g_body(h_ref, d_ref, g_ref):
    g_ref[...] = h_ref[...] * _scale(d_ref)


_g_call = pl.pallas_call(
    _g_body,
    grid=(GRID,),
    in_specs=[
        pl.BlockSpec((BN, F), lambda i: (i, 0)),
        pl.BlockSpec((BN, 1), lambda i: (i, 0)),
    ],
    out_specs=pl.BlockSpec((BN, F), lambda i: (i, 0)),
    out_shape=jax.ShapeDtypeStruct((N, F), jnp.float32),
)


def _layer_body(is_last, h_ref, a0_ref, a1_ref, d_ref, cs_ref,
                wl_ref, fcw_ref, fcb_ref, w1_ref, b1_ref, w2_ref, b2_ref,
                alpha_ref, conv_ref, wo_ref, bo_ref, *outs):
    s = _scale(d_ref)
    hi = (a0_ref[...] + a1_ref[...]) * s
    h = h_ref[...]

    gp = cs_ref[...] * (1.0 / N)
    ge = jnp.dot(jnp.maximum(jnp.dot(gp, w1_ref[...],
                                     preferred_element_type=jnp.float32)
                             + b1_ref[...], 0.0),
                 w2_ref[...], preferred_element_type=jnp.float32) + b2_ref[...]
    wgt = jax.nn.sigmoid(alpha_ref[0, 0])
    m = jnp.dot(wl_ref[...], fcw_ref[...], preferred_element_type=jnp.float32)
    logits = (wgt * jnp.dot(hi, m, preferred_element_type=jnp.float32)
              + ((1.0 - wgt) * jnp.dot(ge, fcw_ref[...],
                                       preferred_element_type=jnp.float32)
                 + fcb_ref[...]))
    e = jax.nn.softmax(logits, axis=-1)

    hcat = jnp.concatenate([hi, h], axis=1)
    outs_all = jnp.dot(hcat, conv_ref[...], preferred_element_type=jnp.float32)
    out = h
    for k in range(K):
        out = out + e[:, k:k + 1] * outs_all[:, k * F:(k + 1) * F]
    hn = jnp.maximum(out, 0.0)

    if is_last:
        outs[0][...] = jnp.dot(hn, wo_ref[...],
                               preferred_element_type=jnp.float32) + bo_ref[...]
    else:
        outs[0][...] = hn
        outs[1][...] = hn * s

        @pl.when(pl.program_id(0) == 0)
        def _():
            outs[2][...] = jnp.zeros_like(outs[2])

        outs[2][...] += jnp.sum(hn, axis=0, keepdims=True)


def _make_layer_call(is_last):
    full = lambda i: (0, 0)
    in_specs = [
        pl.BlockSpec((BN, F), lambda i: (i, 0)),
        pl.BlockSpec((BN, F), lambda i: (i, 0)),
        pl.BlockSpec((BN, F), lambda i: (GRID + i, 0)),
        pl.BlockSpec((BN, 1), lambda i: (i, 0)),
        pl.BlockSpec((1, F), full),
        pl.BlockSpec((F, F), full),
        pl.BlockSpec((F, K), full),
        pl.BlockSpec((1, K), full),
        pl.BlockSpec((F, F), full),
        pl.BlockSpec((1, F), full),
        pl.BlockSpec((F, F), full),
        pl.BlockSpec((1, F), full),
        pl.BlockSpec((1, 1), full),
        pl.BlockSpec((2 * F, K * F), full),
        pl.BlockSpec((F, F), full),
        pl.BlockSpec((1, F), full),
    ]
    if is_last:
        out_specs = [pl.BlockSpec((BN, F), lambda i: (i, 0))]
        out_shape = [jax.ShapeDtypeStruct((N, F), jnp.float32)]
    else:
        out_specs = [
            pl.BlockSpec((BN, F), lambda i: (i, 0)),
            pl.BlockSpec((BN, F), lambda i: (i, 0)),
            pl.BlockSpec((1, F), full),
        ]
        out_shape = [
            jax.ShapeDtypeStruct((N, F), jnp.float32),
            jax.ShapeDtypeStruct((N, F), jnp.float32),
            jax.ShapeDtypeStruct((1, F), jnp.float32),
        ]
    return pl.pallas_call(
        functools.partial(_layer_body, is_last),
        grid=(GRID,),
        in_specs=in_specs,
        out_specs=out_specs,
        out_shape=out_shape,
    )


_layer_call = _make_layer_call(False)
_last_call = _make_layer_call(True)


def kernel(x, edge_index, W_in, b_in, conv_w, env_Wlocal, env_mlp_w1, env_mlp_b1,
           env_mlp_w2, env_mlp_b2, env_alpha, env_fc_w, env_fc_b, W_out, b_out):
    row = edge_index[0].astype(jnp.int32)
    col = edge_index[1].astype(jnp.int32)

    deg = _deg_call(col)
    d = (deg[:N_FLAT] + deg[N_FLAT:])[:N].reshape(N, 1)

    b_in2 = b_in.reshape(1, F)
    b_out2 = b_out.reshape(1, F)

    h, cs = _h_call(x, W_in, b_in2)
    g = _g_call(h, d)

    for l in range(L):
        acc = _spmv_call(row, col, g)
        conv_r = conv_w[l].transpose(1, 0, 2).reshape(2 * F, K * F)
        args = (h, acc, acc, d, cs,
                env_Wlocal[l], env_fc_w[l], env_fc_b[l].reshape(1, K),
                env_mlp_w1[l], env_mlp_b1[l].reshape(1, F),
                env_mlp_w2[l], env_mlp_b2[l].reshape(1, F),
                env_alpha[l].reshape(1, 1), conv_r, W_out, b_out2)
        if l == L - 1:
            out = _last_call(*args)[0]
        else:
            h, g, cs = _layer_call(*args)
    return out
